# Initial kernel scaffold; baseline (speedup 1.0000x reference)
import jax
import jax.numpy as jnp
from jax.experimental import pallas as pl


def kernel(x, edge_index, edge_type, graph_ids, head_ids, tail_ids, rel_labels, spe, W_rel1, W_self1, W_rel2, W_self2, rel_table, W_fc, b_fc, W_out, b_out):
    raise NotImplementedError("write your pallas kernel here")


# speedup vs baseline: 2.8100x; 2.8100x over previous
import functools

import jax
import jax.numpy as jnp
from jax import lax
from jax.experimental import pallas as pl
from jax.experimental.pallas import tpu as pltpu
from jax.experimental.pallas import tpu_sc as plsc

_N = 10000
_E = 320000
_D = 128
_R = 32
_B = 128
_H = 16

_NC = 2
_NS = 16
_HD = _D // 2
_CHUNK = 80
_NCH = _E // (_NS * _CHUNK)
_NPAD = 10240
_RPT = _NPAD // _NS

_BLK = 400
_NBLK = _N // _BLK


def _sc_agg_body(with_deg, *refs):
    if with_deg:
        (t_hbm, src_hbm, ety_hbm, dst_hbm,
         oa0, oa1, od,
         src_v, ety_v, dst_v, row_v, z_v, agg_sh, sem,
         ones_v, zd_v, deg_sh) = refs
    else:
        (t_hbm, src_hbm, ety_hbm, dst_hbm,
         oa0, oa1,
         src_v, ety_v, dst_v, row_v, z_v, agg_sh, sem) = refs

    cid = lax.axis_index("c")
    sid = lax.axis_index("s")

    pltpu.sync_copy(src_hbm.at[sid], src_v)
    pltpu.sync_copy(ety_hbm.at[sid], ety_v)
    pltpu.sync_copy(dst_hbm.at[sid], dst_v)

    def idx_body(r, carry):
        for j in range(_CHUNK // 16):
            sl = pl.ds(j * 16, 16)
            ety_v[r, sl] = src_v[r, sl] * (2 * _R) + ety_v[r, sl] * 2 + cid
        return carry
    lax.fori_loop(0, _NCH, idx_body, 0)

    def z_body(r, carry):
        for j in range(_HD // 16):
            z_v[r, pl.ds(j * 16, 16)] = jnp.zeros((16,), jnp.float32)
        return carry
    lax.fori_loop(0, 128, z_body, 0)
    for k in range(_RPT // 128):
        pltpu.sync_copy(z_v, agg_sh.at[pl.ds(sid * _RPT + k * 128, 128)])

    if with_deg:
        def o_body(r, carry):
            ones_v[r, pl.ds(0, 16)] = jnp.ones((16,), jnp.float32)
            return carry
        lax.fori_loop(0, _CHUNK, o_body, 0)
        def zd_body(r, carry):
            zd_v[r, pl.ds(0, 16)] = jnp.zeros((16,), jnp.float32)
            return carry
        lax.fori_loop(0, 128, zd_body, 0)
        for k in range(_RPT // 128):
            pltpu.sync_copy(zd_v, deg_sh.at[pl.ds(sid * _RPT + k * 128, 128)])

    plsc.subcore_barrier()

    def chunk_body(c, carry):
        pltpu.async_copy(t_hbm.at[ety_v.at[c]], row_v, sem).wait()
        pltpu.sync_copy(row_v, agg_sh.at[dst_v.at[c]], add=True)
        if with_deg:
            @pl.when(cid == 0)
            def _():
                pltpu.sync_copy(ones_v, deg_sh.at[dst_v.at[c]], add=True)
        return carry
    lax.fori_loop(0, _NCH, chunk_body, 0)

    plsc.subcore_barrier()

    rs = pl.ds(sid * _RPT, _RPT)

    @pl.when(cid == 0)
    def _():
        pltpu.sync_copy(agg_sh.at[rs], oa0.at[rs])
        if with_deg:
            pltpu.sync_copy(deg_sh.at[rs], od.at[rs])

    @pl.when(cid == 1)
    def _():
        pltpu.sync_copy(agg_sh.at[rs], oa1.at[rs])


def _make_sc_agg(with_deg):
    out_type = [jax.ShapeDtypeStruct((_NPAD, _HD), jnp.float32),
                jax.ShapeDtypeStruct((_NPAD, _HD), jnp.float32)]
    scratch = [
        pltpu.VMEM((_NCH, _CHUNK), jnp.int32),
        pltpu.VMEM((_NCH, _CHUNK), jnp.int32),
        pltpu.VMEM((_NCH, _CHUNK), jnp.int32),
        pltpu.VMEM((_CHUNK, _HD), jnp.float32),
        pltpu.VMEM((128, _HD), jnp.float32),
        pltpu.VMEM_SHARED((_NPAD, _HD), jnp.float32),
        pltpu.SemaphoreType.DMA,
    ]
    if with_deg:
        out_type += [jax.ShapeDtypeStruct((_NPAD, 16), jnp.float32)]
        scratch += [
            pltpu.VMEM((_CHUNK, 16), jnp.float32),
            pltpu.VMEM((128, 16), jnp.float32),
            pltpu.VMEM_SHARED((_NPAD, 16), jnp.float32),
        ]
    mesh = plsc.VectorSubcoreMesh(core_axis_name="c", subcore_axis_name="s")
    return pl.kernel(functools.partial(_sc_agg_body, with_deg),
                     out_type=tuple(out_type), mesh=mesh,
                     scratch_types=scratch,
                     compiler_params=pltpu.CompilerParams(
                         use_tc_tiling_on_sc=False))


_sc_agg_deg = _make_sc_agg(True)
_sc_agg = _make_sc_agg(False)


def _transform_body(h_ref, wf_ref, ws_ref, t_ref, s_ref):
    hb = h_ref[...]
    t_ref[...] = jnp.dot(hb, wf_ref[...], preferred_element_type=jnp.float32)
    s_ref[...] = jnp.dot(hb, ws_ref[...], preferred_element_type=jnp.float32)


def _transform(h, wflat, wself):
    return pl.pallas_call(
        _transform_body,
        grid=(_NBLK,),
        in_specs=[pl.BlockSpec((_BLK, _D), lambda i: (i, 0)),
                  pl.BlockSpec((_D, _R * _D), lambda i: (0, 0)),
                  pl.BlockSpec((_D, _D), lambda i: (0, 0))],
        out_specs=[pl.BlockSpec((_BLK, _R * _D), lambda i: (i, 0)),
                   pl.BlockSpec((_BLK, _D), lambda i: (i, 0))],
        out_shape=[jax.ShapeDtypeStruct((_N, _R * _D), jnp.float32),
                   jax.ShapeDtypeStruct((_N, _D), jnp.float32)],
    )(h, wflat, wself)


def _combine_body(a0_ref, a1_ref, d_ref, s_ref, wf_ref, ws_ref,
                  t_ref, s2_ref, h_ref):
    deg = d_ref[...][:, 0:1]
    agg = jnp.concatenate([a0_ref[...], a1_ref[...]], axis=1)
    h1 = jnp.maximum(agg / jnp.maximum(deg, 1.0) + s_ref[...], 0.0)
    h_ref[...] = h1
    t_ref[...] = jnp.dot(h1, wf_ref[...], preferred_element_type=jnp.float32)
    s2_ref[...] = jnp.dot(h1, ws_ref[...], preferred_element_type=jnp.float32)


def _combine(a0, a1, d, s1, wflat, wself):
    return pl.pallas_call(
        _combine_body,
        grid=(_NBLK,),
        in_specs=[pl.BlockSpec((_BLK, _HD), lambda i: (i, 0)),
                  pl.BlockSpec((_BLK, _HD), lambda i: (i, 0)),
                  pl.BlockSpec((_BLK, 16), lambda i: (i, 0)),
                  pl.BlockSpec((_BLK, _D), lambda i: (i, 0)),
                  pl.BlockSpec((_D, _R * _D), lambda i: (0, 0)),
                  pl.BlockSpec((_D, _D), lambda i: (0, 0))],
        out_specs=[pl.BlockSpec((_BLK, _R * _D), lambda i: (i, 0)),
                   pl.BlockSpec((_BLK, _D), lambda i: (i, 0)),
                   pl.BlockSpec((_BLK, _D), lambda i: (i, 0))],
        out_shape=[jax.ShapeDtypeStruct((_N, _R * _D), jnp.float32),
                   jax.ShapeDtypeStruct((_N, _D), jnp.float32),
                   jax.ShapeDtypeStruct((_N, _D), jnp.float32)],
    )(a0, a1, d, s1, wflat, wself)


def _readout_body(a0_ref, a1_ref, d_ref, s2_ref, h1_ref, gid_ref,
                  hid_ref, tid_ref, rel_ref, spe_ref, rt_ref,
                  wg_ref, wh_ref, wt_ref, wr_ref, ws_ref, bfc_ref, wo_ref,
                  bo_ref, out_ref, G, HD, TL, CNT):
    i = pl.program_id(0)

    @pl.when(i == 0)
    def _():
        G[...] = jnp.zeros_like(G)
        HD[...] = jnp.zeros_like(HD)
        TL[...] = jnp.zeros_like(TL)
        CNT[...] = jnp.zeros_like(CNT)

    deg = d_ref[...][:, 0:1]
    agg = jnp.concatenate([a0_ref[...], a1_ref[...]], axis=1)
    h2 = jnp.maximum(agg / jnp.maximum(deg, 1.0) + s2_ref[...], 0.0)
    flat = jnp.concatenate([h1_ref[...], h2], axis=1)

    gb = gid_ref[0, 0, :]
    og = (gb[None, :] == lax.broadcasted_iota(jnp.int32, (_B, _BLK), 0)
          ).astype(jnp.float32)
    G[...] += jnp.dot(og, flat, preferred_element_type=jnp.float32)
    CNT[...] = CNT[...] + jnp.sum(og, axis=1, keepdims=True)

    rowid = lax.broadcasted_iota(jnp.int32, (_B, _BLK), 1) + i * _BLK
    oh = (hid_ref[0, :][:, None] == rowid).astype(jnp.float32)
    ot = (tid_ref[0, :][:, None] == rowid).astype(jnp.float32)
    HD[...] += jnp.dot(oh, flat, preferred_element_type=jnp.float32)
    TL[...] += jnp.dot(ot, flat, preferred_element_type=jnp.float32)

    @pl.when(i == _NBLK - 1)
    def _():
        g_out = G[...] / jnp.maximum(CNT[...][:, 0:1], 1.0)
        orl = (rel_ref[0, :][:, None] ==
               lax.broadcasted_iota(jnp.int32, (_B, _R), 1)).astype(jnp.float32)
        relemb = jnp.dot(orl, rt_ref[...], preferred_element_type=jnp.float32)
        hid = (jnp.dot(g_out, wg_ref[...], preferred_element_type=jnp.float32)
               + jnp.dot(HD[...], wh_ref[...], preferred_element_type=jnp.float32)
               + jnp.dot(TL[...], wt_ref[...], preferred_element_type=jnp.float32)
               + jnp.dot(relemb, wr_ref[...], preferred_element_type=jnp.float32)
               + jnp.dot(spe_ref[...], ws_ref[...], preferred_element_type=jnp.float32)
               + bfc_ref[...])
        hid = jnp.maximum(hid, 0.0)
        res = jnp.sum(hid * wo_ref[...], axis=1, keepdims=True) + bo_ref[0, 0]
        out_ref[...] = jnp.broadcast_to(res, (_B, _D))


def _readout(a0, a1, d, s2, h1, gid3, hid2, tid2, rel2, spe, rtab,
             wg, wh, wt, wr, ws, bfc2, wo2, bo2):
    def cst(*dims):
        return pl.BlockSpec(dims, lambda i: tuple(0 for _ in dims))
    return pl.pallas_call(
        _readout_body,
        grid=(_NBLK,),
        in_specs=[pl.BlockSpec((_BLK, _HD), lambda i: (i, 0)),
                  pl.BlockSpec((_BLK, _HD), lambda i: (i, 0)),
                  pl.BlockSpec((_BLK, 16), lambda i: (i, 0)),
                  pl.BlockSpec((_BLK, _D), lambda i: (i, 0)),
                  pl.BlockSpec((_BLK, _D), lambda i: (i, 0)),
                  pl.BlockSpec((1, 1, _BLK), lambda i: (i, 0, 0)),
                  cst(1, _B), cst(1, _B), cst(1, _B),
                  cst(_B, 16), cst(_R, _R),
                  cst(2 * _D, _H), cst(2 * _D, _H), cst(2 * _D, _H),
                  cst(_R, _H), cst(16, _H), cst(1, _H), cst(1, _H),
                  cst(1, 1)],
        out_specs=pl.BlockSpec((_B, _D), lambda i: (0, 0)),
        out_shape=jax.ShapeDtypeStruct((_B, _D), jnp.float32),
        scratch_shapes=[pltpu.VMEM((_B, 2 * _D), jnp.float32),
                        pltpu.VMEM((_B, 2 * _D), jnp.float32),
                        pltpu.VMEM((_B, 2 * _D), jnp.float32),
                        pltpu.VMEM((_B, _D), jnp.float32)],
    )(a0, a1, d, s2, h1, gid3, hid2, tid2, rel2, spe, rtab,
      wg, wh, wt, wr, ws, bfc2, wo2, bo2)


def kernel(x, edge_index, edge_type, graph_ids, head_ids, tail_ids,
           rel_labels, spe, W_rel1, W_self1, W_rel2, W_self2, rel_table,
           W_fc, b_fc, W_out, b_out):
    src3 = edge_index[0].astype(jnp.int32).reshape(_NS, _NCH, _CHUNK)
    dst3 = edge_index[1].astype(jnp.int32).reshape(_NS, _NCH, _CHUNK)
    ety3 = edge_type.astype(jnp.int32).reshape(_NS, _NCH, _CHUNK)
    wf1 = jnp.transpose(W_rel1, (1, 0, 2)).reshape(_D, _R * _D)
    wf2 = jnp.transpose(W_rel2, (1, 0, 2)).reshape(_D, _R * _D)

    T1, S1 = _transform(x, wf1, W_self1)
    a0, a1, d = _sc_agg_deg(T1.reshape(_N * _R * 2, _HD), src3, ety3, dst3)
    T2, S2, H1 = _combine(a0, a1, d, S1, wf2, W_self2)
    b0, b1 = _sc_agg(T2.reshape(_N * _R * 2, _HD), src3, ety3, dst3)

    OUT = _readout(b0, b1, d, S2, H1,
                   graph_ids.astype(jnp.int32).reshape(_NBLK, 1, _BLK),
                   head_ids.astype(jnp.int32).reshape(1, _B),
                   tail_ids.astype(jnp.int32).reshape(1, _B),
                   rel_labels.astype(jnp.int32).reshape(1, _B),
                   spe, rel_table,
                   W_fc[0:2 * _D], W_fc[2 * _D:4 * _D], W_fc[4 * _D:6 * _D],
                   W_fc[6 * _D:6 * _D + _R], W_fc[6 * _D + _R:],
                   b_fc.reshape(1, _H), W_out.reshape(1, _H),
                   b_out.reshape(1, 1))
    return OUT[:, :1]


# speedup vs baseline: 3.5652x; 1.2687x over previous
import functools

import jax
import jax.numpy as jnp
from jax import lax
from jax.experimental import pallas as pl
from jax.experimental.pallas import tpu as pltpu
from jax.experimental.pallas import tpu_sc as plsc

_N = 10000
_E = 320000
_D = 128
_R = 32
_B = 128
_H = 16

_NC = 2
_NS = 16
_HD = _D // 2
_CHUNK = 80
_NCH = _E // (_NS * _CHUNK)
_RPT = 632
_RPTL = _N - 15 * _RPT

_BLK = 400
_NBLK = _N // _BLK


def _sc_agg_body(with_deg, *refs):
    if with_deg:
        (t_hbm, src_hbm, ety_hbm, dst_hbm,
         oa0, oa1, od0, od1,
         src_v, ety_v, dst_v, rowa_v, rowb_v, z_v, agg_sh, sema, semb,
         ones_v, zd_v, deg_sh) = refs
    else:
        (t_hbm, src_hbm, ety_hbm, dst_hbm,
         oa0, oa1,
         src_v, ety_v, dst_v, rowa_v, rowb_v, z_v, agg_sh, sema, semb) = refs

    cid = lax.axis_index("c")
    sid = lax.axis_index("s")

    pltpu.sync_copy(src_hbm.at[sid], src_v)
    pltpu.sync_copy(ety_hbm.at[sid], ety_v)
    pltpu.sync_copy(dst_hbm.at[sid], dst_v)

    def idx_body(r, carry):
        for j in range(_CHUNK // 16):
            sl = pl.ds(j * 16, 16)
            ety_v[r, sl] = ety_v[r, sl] * (2 * _N) + src_v[r, sl] * 2 + cid
        return carry
    lax.fori_loop(0, _NCH, idx_body, 0)

    def z_body(r, carry):
        for j in range(_HD // 16):
            z_v[r, pl.ds(j * 16, 16)] = jnp.zeros((16,), jnp.float32)
        return carry
    lax.fori_loop(0, 128, z_body, 0)
    base = sid * _RPT

    @pl.when(sid < 15)
    def _():
        for k in range(4):
            pltpu.sync_copy(z_v, agg_sh.at[pl.ds(base + k * 128, 128)])
        pltpu.sync_copy(z_v.at[pl.ds(0, _RPT - 512)],
                        agg_sh.at[pl.ds(base + 512, _RPT - 512)])

    @pl.when(sid == 15)
    def _():
        for k in range(4):
            pltpu.sync_copy(z_v, agg_sh.at[pl.ds(base + k * 128, 128)])
        pltpu.sync_copy(z_v.at[pl.ds(0, _RPTL - 512)],
                        agg_sh.at[pl.ds(base + 512, _RPTL - 512)])

    if with_deg:
        def o_body(r, carry):
            ones_v[r, pl.ds(0, 16)] = jnp.ones((16,), jnp.float32)
            return carry
        lax.fori_loop(0, _CHUNK, o_body, 0)
        def zd_body(r, carry):
            zd_v[r, pl.ds(0, 16)] = jnp.zeros((16,), jnp.float32)
            return carry
        lax.fori_loop(0, 128, zd_body, 0)

        @pl.when(sid < 15)
        def _():
            for k in range(4):
                pltpu.sync_copy(zd_v, deg_sh.at[pl.ds(base + k * 128, 128)])
            pltpu.sync_copy(zd_v.at[pl.ds(0, _RPT - 512)],
                            deg_sh.at[pl.ds(base + 512, _RPT - 512)])

        @pl.when(sid == 15)
        def _():
            for k in range(4):
                pltpu.sync_copy(zd_v, deg_sh.at[pl.ds(base + k * 128, 128)])
            pltpu.sync_copy(zd_v.at[pl.ds(0, _RPTL - 512)],
                            deg_sh.at[pl.ds(base + 512, _RPTL - 512)])

    plsc.subcore_barrier()

    def scat(row_v, c):
        pltpu.sync_copy(row_v, agg_sh.at[dst_v.at[c]], add=True)
        if with_deg:
            @pl.when(cid == (c >= _NCH // 2).astype(jnp.int32))
            def _():
                pltpu.sync_copy(ones_v, deg_sh.at[dst_v.at[c]], add=True)

    def chunk_body(c, carry):
        pltpu.async_copy(t_hbm.at[ety_v.at[c]], rowa_v, sema).wait()
        scat(rowa_v, c)
        return carry
    lax.fori_loop(0, _NCH, chunk_body, 0)

    plsc.subcore_barrier()

    rs = pl.ds(base, _RPT)
    rsl = pl.ds(base, _RPTL)

    @pl.when(jnp.logical_and(cid == 0, sid < 15))
    def _():
        pltpu.sync_copy(agg_sh.at[rs], oa0.at[rs])
        if with_deg:
            pltpu.sync_copy(deg_sh.at[rs], od0.at[rs])

    @pl.when(jnp.logical_and(cid == 0, sid == 15))
    def _():
        pltpu.sync_copy(agg_sh.at[rsl], oa0.at[rsl])
        if with_deg:
            pltpu.sync_copy(deg_sh.at[rsl], od0.at[rsl])

    @pl.when(jnp.logical_and(cid == 1, sid < 15))
    def _():
        pltpu.sync_copy(agg_sh.at[rs], oa1.at[rs])
        if with_deg:
            pltpu.sync_copy(deg_sh.at[rs], od1.at[rs])

    @pl.when(jnp.logical_and(cid == 1, sid == 15))
    def _():
        pltpu.sync_copy(agg_sh.at[rsl], oa1.at[rsl])
        if with_deg:
            pltpu.sync_copy(deg_sh.at[rsl], od1.at[rsl])


def _make_sc_agg(with_deg):
    out_type = [jax.ShapeDtypeStruct((_N, _HD), jnp.float32),
                jax.ShapeDtypeStruct((_N, _HD), jnp.float32)]
    scratch = [
        pltpu.VMEM((_NCH, _CHUNK), jnp.int32),
        pltpu.VMEM((_NCH, _CHUNK), jnp.int32),
        pltpu.VMEM((_NCH, _CHUNK), jnp.int32),
        pltpu.VMEM((_CHUNK, _HD), jnp.float32),
        pltpu.VMEM((_CHUNK, _HD), jnp.float32),
        pltpu.VMEM((128, _HD), jnp.float32),
        pltpu.VMEM_SHARED((_N, _HD), jnp.float32),
        pltpu.SemaphoreType.DMA,
        pltpu.SemaphoreType.DMA,
    ]
    if with_deg:
        out_type += [jax.ShapeDtypeStruct((_N, 16), jnp.float32),
                     jax.ShapeDtypeStruct((_N, 16), jnp.float32)]
        scratch += [
            pltpu.VMEM((_CHUNK, 16), jnp.float32),
            pltpu.VMEM((128, 16), jnp.float32),
            pltpu.VMEM_SHARED((_N, 16), jnp.float32),
        ]
    mesh = plsc.VectorSubcoreMesh(core_axis_name="c", subcore_axis_name="s")
    return pl.kernel(functools.partial(_sc_agg_body, with_deg),
                     out_type=tuple(out_type), mesh=mesh,
                     scratch_types=scratch,
                     compiler_params=pltpu.CompilerParams(
                         use_tc_tiling_on_sc=False))


_sc_agg_deg = _make_sc_agg(True)
_sc_agg = _make_sc_agg(False)


def _transform_body(h_ref, wr_ref, ws_ref, t_ref, s_ref):
    hb = h_ref[...]
    hb16 = hb.astype(jnp.bfloat16)
    for r in range(_R):
        t_ref[r] = jnp.dot(hb16, wr_ref[r], preferred_element_type=jnp.float32)
    s_ref[...] = jnp.dot(hb, ws_ref[...], preferred_element_type=jnp.float32)


def _transform(h, wrel, wself):
    return pl.pallas_call(
        _transform_body,
        grid=(_NBLK,),
        in_specs=[pl.BlockSpec((_BLK, _D), lambda i: (i, 0)),
                  pl.BlockSpec((_R, _D, _D), lambda i: (0, 0, 0)),
                  pl.BlockSpec((_D, _D), lambda i: (0, 0))],
        out_specs=[pl.BlockSpec((_R, _BLK, _D), lambda i: (0, i, 0)),
                   pl.BlockSpec((_BLK, _D), lambda i: (i, 0))],
        out_shape=[jax.ShapeDtypeStruct((_R, _N, _D), jnp.float32),
                   jax.ShapeDtypeStruct((_N, _D), jnp.float32)],
    )(h, wrel, wself)


def _combine_body(a0_ref, a1_ref, d0_ref, d1_ref, s_ref, wr_ref, ws_ref,
                  t_ref, s2_ref, h_ref):
    deg = d0_ref[...][:, 0:1] + d1_ref[...][:, 0:1]
    agg = jnp.concatenate([a0_ref[...], a1_ref[...]], axis=1)
    h1 = jnp.maximum(agg / jnp.maximum(deg, 1.0) + s_ref[...], 0.0)
    h_ref[...] = h1
    s2_ref[...] = jnp.dot(h1, ws_ref[...], preferred_element_type=jnp.float32)
    h16 = h1.astype(jnp.bfloat16)
    for r in range(_R):
        t_ref[r] = jnp.dot(h16, wr_ref[r], preferred_element_type=jnp.float32)


def _combine(a0, a1, d0, d1, s1, wrel, wself):
    return pl.pallas_call(
        _combine_body,
        grid=(_NBLK,),
        in_specs=[pl.BlockSpec((_BLK, _HD), lambda i: (i, 0)),
                  pl.BlockSpec((_BLK, _HD), lambda i: (i, 0)),
                  pl.BlockSpec((_BLK, 16), lambda i: (i, 0)),
                  pl.BlockSpec((_BLK, 16), lambda i: (i, 0)),
                  pl.BlockSpec((_BLK, _D), lambda i: (i, 0)),
                  pl.BlockSpec((_R, _D, _D), lambda i: (0, 0, 0)),
                  pl.BlockSpec((_D, _D), lambda i: (0, 0))],
        out_specs=[pl.BlockSpec((_R, _BLK, _D), lambda i: (0, i, 0)),
                   pl.BlockSpec((_BLK, _D), lambda i: (i, 0)),
                   pl.BlockSpec((_BLK, _D), lambda i: (i, 0))],
        out_shape=[jax.ShapeDtypeStruct((_R, _N, _D), jnp.float32),
                   jax.ShapeDtypeStruct((_N, _D), jnp.float32),
                   jax.ShapeDtypeStruct((_N, _D), jnp.float32)],
    )(a0, a1, d0, d1, s1, wrel, wself)


def _readout_body(a0_ref, a1_ref, d0_ref, d1_ref, s2_ref, h1_ref, gid_ref,
                  hid_ref, tid_ref, rel_ref, spe_ref, rt_ref,
                  wg_ref, wh_ref, wt_ref, wr_ref, ws_ref, bfc_ref, wo_ref,
                  bo_ref, out_ref, G, HD, TL, CNT):
    i = pl.program_id(0)

    @pl.when(i == 0)
    def _():
        G[...] = jnp.zeros_like(G)
        HD[...] = jnp.zeros_like(HD)
        TL[...] = jnp.zeros_like(TL)
        CNT[...] = jnp.zeros_like(CNT)

    deg = d0_ref[...][:, 0:1] + d1_ref[...][:, 0:1]
    agg = jnp.concatenate([a0_ref[...], a1_ref[...]], axis=1)
    h2 = jnp.maximum(agg / jnp.maximum(deg, 1.0) + s2_ref[...], 0.0)
    flat = jnp.concatenate([h1_ref[...], h2], axis=1)

    gb = gid_ref[0, 0, :]
    og = (gb[None, :] == lax.broadcasted_iota(jnp.int32, (_B, _BLK), 0)
          ).astype(jnp.float32)
    G[...] += jnp.dot(og, flat, preferred_element_type=jnp.float32)
    CNT[...] = CNT[...] + jnp.sum(og, axis=1, keepdims=True)

    rowid = lax.broadcasted_iota(jnp.int32, (_B, _BLK), 1) + i * _BLK
    oh = (hid_ref[0, :][:, None] == rowid).astype(jnp.float32)
    ot = (tid_ref[0, :][:, None] == rowid).astype(jnp.float32)
    HD[...] += jnp.dot(oh, flat, preferred_element_type=jnp.float32)
    TL[...] += jnp.dot(ot, flat, preferred_element_type=jnp.float32)

    @pl.when(i == _NBLK - 1)
    def _():
        g_out = G[...] / jnp.maximum(CNT[...][:, 0:1], 1.0)
        orl = (rel_ref[0, :][:, None] ==
               lax.broadcasted_iota(jnp.int32, (_B, _R), 1)).astype(jnp.float32)
        relemb = jnp.dot(orl, rt_ref[...], preferred_element_type=jnp.float32)
        hid = (jnp.dot(g_out, wg_ref[...], preferred_element_type=jnp.float32)
               + jnp.dot(HD[...], wh_ref[...], preferred_element_type=jnp.float32)
               + jnp.dot(TL[...], wt_ref[...], preferred_element_type=jnp.float32)
               + jnp.dot(relemb, wr_ref[...], preferred_element_type=jnp.float32)
               + jnp.dot(spe_ref[...], ws_ref[...], preferred_element_type=jnp.float32)
               + bfc_ref[...])
        hid = jnp.maximum(hid, 0.0)
        res = jnp.sum(hid * wo_ref[...], axis=1, keepdims=True) + bo_ref[0, 0]
        out_ref[...] = jnp.broadcast_to(res, (_B, _D))


def _readout(a0, a1, d0, d1, s2, h1, gid3, hid2, tid2, rel2, spe, rtab,
             wg, wh, wt, wr, ws, bfc2, wo2, bo2):
    def cst(*dims):
        return pl.BlockSpec(dims, lambda i: tuple(0 for _ in dims))
    return pl.pallas_call(
        _readout_body,
        grid=(_NBLK,),
        in_specs=[pl.BlockSpec((_BLK, _HD), lambda i: (i, 0)),
                  pl.BlockSpec((_BLK, _HD), lambda i: (i, 0)),
                  pl.BlockSpec((_BLK, 16), lambda i: (i, 0)),
                  pl.BlockSpec((_BLK, 16), lambda i: (i, 0)),
                  pl.BlockSpec((_BLK, _D), lambda i: (i, 0)),
                  pl.BlockSpec((_BLK, _D), lambda i: (i, 0)),
                  pl.BlockSpec((1, 1, _BLK), lambda i: (i, 0, 0)),
                  cst(1, _B), cst(1, _B), cst(1, _B),
                  cst(_B, 16), cst(_R, _R),
                  cst(2 * _D, _H), cst(2 * _D, _H), cst(2 * _D, _H),
                  cst(_R, _H), cst(16, _H), cst(1, _H), cst(1, _H),
                  cst(1, 1)],
        out_specs=pl.BlockSpec((_B, _D), lambda i: (0, 0)),
        out_shape=jax.ShapeDtypeStruct((_B, _D), jnp.float32),
        scratch_shapes=[pltpu.VMEM((_B, 2 * _D), jnp.float32),
                        pltpu.VMEM((_B, 2 * _D), jnp.float32),
                        pltpu.VMEM((_B, 2 * _D), jnp.float32),
                        pltpu.VMEM((_B, _D), jnp.float32)],
    )(a0, a1, d0, d1, s2, h1, gid3, hid2, tid2, rel2, spe, rtab,
      wg, wh, wt, wr, ws, bfc2, wo2, bo2)


def kernel(x, edge_index, edge_type, graph_ids, head_ids, tail_ids,
           rel_labels, spe, W_rel1, W_self1, W_rel2, W_self2, rel_table,
           W_fc, b_fc, W_out, b_out):
    src3 = edge_index[0].astype(jnp.int32).reshape(_NS, _NCH, _CHUNK)
    dst3 = edge_index[1].astype(jnp.int32).reshape(_NS, _NCH, _CHUNK)
    ety3 = edge_type.astype(jnp.int32).reshape(_NS, _NCH, _CHUNK)

    T1, S1 = _transform(x, W_rel1.astype(jnp.bfloat16), W_self1)
    a0, a1, d0, d1 = _sc_agg_deg(T1.reshape(_N * _R * 2, _HD),
                                 src3, ety3, dst3)
    T2, S2, H1 = _combine(a0, a1, d0, d1, S1,
                          W_rel2.astype(jnp.bfloat16), W_self2)
    b0, b1 = _sc_agg(T2.reshape(_N * _R * 2, _HD), src3, ety3, dst3)

    OUT = _readout(b0, b1, d0, d1, S2, H1,
                   graph_ids.astype(jnp.int32).reshape(_NBLK, 1, _BLK),
                   head_ids.astype(jnp.int32).reshape(1, _B),
                   tail_ids.astype(jnp.int32).reshape(1, _B),
                   rel_labels.astype(jnp.int32).reshape(1, _B),
                   spe, rel_table,
                   W_fc[0:2 * _D], W_fc[2 * _D:4 * _D], W_fc[4 * _D:6 * _D],
                   W_fc[6 * _D:6 * _D + _R], W_fc[6 * _D + _R:],
                   b_fc.reshape(1, _H), W_out.reshape(1, _H),
                   b_out.reshape(1, 1))
    return OUT[:, :1]


# speedup vs baseline: 4.2791x; 1.2002x over previous
import functools

import jax
import jax.numpy as jnp
from jax import lax
from jax.experimental import pallas as pl
from jax.experimental.pallas import tpu as pltpu
from jax.experimental.pallas import tpu_sc as plsc

_N = 10000
_E = 320000
_D = 128
_R = 32
_B = 128
_H = 16

_NC = 2
_NS = 16
_HD = _D // 2
_CHUNK = 80
_NCH = _E // (_NS * _CHUNK)
_RPT = 632
_RPTL = _N - 15 * _RPT

_BLK = 400
_NBLK = _N // _BLK


def _sc_agg_body(with_deg, *refs):
    if with_deg:
        (t_hbm, src_hbm, ety_hbm, dst_hbm, ones8_hbm, zer8_hbm,
         oa0, oa1, od0, od1,
         src_v, ety_v, dst_v, rowa_v, rowb_v, z_v, agg_sh, sema, semb,
         ones_v, zd_v, deg_sh) = refs
    else:
        (t_hbm, src_hbm, ety_hbm, dst_hbm,
         oa0, oa1,
         src_v, ety_v, dst_v, rowa_v, rowb_v, z_v, agg_sh, sema, semb) = refs

    cid = lax.axis_index("c")
    sid = lax.axis_index("s")

    pltpu.sync_copy(src_hbm.at[sid], src_v)
    pltpu.sync_copy(ety_hbm.at[sid], ety_v)
    pltpu.sync_copy(dst_hbm.at[sid], dst_v)

    def idx_body(r, carry):
        for j in range(_CHUNK // 16):
            sl = pl.ds(j * 16, 16)
            ety_v[r, sl] = ety_v[r, sl] * (2 * _N) + src_v[r, sl] * 2 + cid
        return carry
    lax.fori_loop(0, _NCH, idx_body, 0)

    def z_body(r, carry):
        for j in range(_HD // 16):
            z_v[r, pl.ds(j * 16, 16)] = jnp.zeros((16,), jnp.float32)
        return carry
    lax.fori_loop(0, 128, z_body, 0)
    base = sid * _RPT

    @pl.when(sid < 15)
    def _():
        for k in range(4):
            pltpu.sync_copy(z_v, agg_sh.at[pl.ds(base + k * 128, 128)])
        pltpu.sync_copy(z_v.at[pl.ds(0, _RPT - 512)],
                        agg_sh.at[pl.ds(base + 512, _RPT - 512)])

    @pl.when(sid == 15)
    def _():
        for k in range(4):
            pltpu.sync_copy(z_v, agg_sh.at[pl.ds(base + k * 128, 128)])
        pltpu.sync_copy(z_v.at[pl.ds(0, _RPTL - 512)],
                        agg_sh.at[pl.ds(base + 512, _RPTL - 512)])

    if with_deg:
        pltpu.sync_copy(ones8_hbm, ones_v)
        pltpu.sync_copy(zer8_hbm, zd_v)

        @pl.when(sid < 15)
        def _():
            for k in range(4):
                pltpu.sync_copy(zd_v, deg_sh.at[pl.ds(base + k * 128, 128)])
            pltpu.sync_copy(zd_v.at[pl.ds(0, _RPT - 512)],
                            deg_sh.at[pl.ds(base + 512, _RPT - 512)])

        @pl.when(sid == 15)
        def _():
            for k in range(4):
                pltpu.sync_copy(zd_v, deg_sh.at[pl.ds(base + k * 128, 128)])
            pltpu.sync_copy(zd_v.at[pl.ds(0, _RPTL - 512)],
                            deg_sh.at[pl.ds(base + 512, _RPTL - 512)])

    plsc.subcore_barrier()

    def scat(row_v, c):
        pltpu.sync_copy(row_v, agg_sh.at[dst_v.at[c]], add=True)
        if with_deg:
            @pl.when(cid == (c >= _NCH // 2).astype(jnp.int32))
            def _():
                pltpu.sync_copy(ones_v, deg_sh.at[dst_v.at[c]], add=True)

    pltpu.async_copy(t_hbm.at[ety_v.at[0]], rowa_v, sema)

    def pair_body(p, carry):
        c0 = 2 * p
        pltpu.make_async_copy(t_hbm.at[ety_v.at[c0]], rowa_v, sema).wait()
        pltpu.async_copy(t_hbm.at[ety_v.at[c0 + 1]], rowb_v, semb)
        scat(rowa_v, c0)
        pltpu.make_async_copy(t_hbm.at[ety_v.at[c0 + 1]], rowb_v, semb).wait()

        @pl.when(c0 + 2 < _NCH)
        def _():
            pltpu.async_copy(t_hbm.at[ety_v.at[c0 + 2]], rowa_v, sema)

        scat(rowb_v, c0 + 1)
        return carry
    lax.fori_loop(0, _NCH // 2, pair_body, 0)

    plsc.subcore_barrier()

    rs = pl.ds(base, _RPT)
    rsl = pl.ds(base, _RPTL)

    @pl.when(jnp.logical_and(cid == 0, sid < 15))
    def _():
        pltpu.sync_copy(agg_sh.at[rs], oa0.at[rs])
        if with_deg:
            pltpu.sync_copy(deg_sh.at[rs], od0.at[rs])

    @pl.when(jnp.logical_and(cid == 0, sid == 15))
    def _():
        pltpu.sync_copy(agg_sh.at[rsl], oa0.at[rsl])
        if with_deg:
            pltpu.sync_copy(deg_sh.at[rsl], od0.at[rsl])

    @pl.when(jnp.logical_and(cid == 1, sid < 15))
    def _():
        pltpu.sync_copy(agg_sh.at[rs], oa1.at[rs])
        if with_deg:
            pltpu.sync_copy(deg_sh.at[rs], od1.at[rs])

    @pl.when(jnp.logical_and(cid == 1, sid == 15))
    def _():
        pltpu.sync_copy(agg_sh.at[rsl], oa1.at[rsl])
        if with_deg:
            pltpu.sync_copy(deg_sh.at[rsl], od1.at[rsl])


def _make_sc_agg(with_deg):
    out_type = [jax.ShapeDtypeStruct((_N, _HD), jnp.float32),
                jax.ShapeDtypeStruct((_N, _HD), jnp.float32)]
    scratch = [
        pltpu.VMEM((_NCH, _CHUNK), jnp.int32),
        pltpu.VMEM((_NCH, _CHUNK), jnp.int32),
        pltpu.VMEM((_NCH, _CHUNK), jnp.int32),
        pltpu.VMEM((_CHUNK, _HD), jnp.float32),
        pltpu.VMEM((_CHUNK, _HD), jnp.float32),
        pltpu.VMEM((128, _HD), jnp.float32),
        pltpu.VMEM_SHARED((_N, _HD), jnp.float32),
        pltpu.SemaphoreType.DMA,
        pltpu.SemaphoreType.DMA,
    ]
    if with_deg:
        out_type += [jax.ShapeDtypeStruct((_N, 8), jnp.float32),
                     jax.ShapeDtypeStruct((_N, 8), jnp.float32)]
        scratch += [
            pltpu.VMEM((_CHUNK, 8), jnp.float32),
            pltpu.VMEM((128, 8), jnp.float32),
            pltpu.VMEM_SHARED((_N, 8), jnp.float32),
        ]
    mesh = plsc.VectorSubcoreMesh(core_axis_name="c", subcore_axis_name="s")
    return pl.kernel(functools.partial(_sc_agg_body, with_deg),
                     out_type=tuple(out_type), mesh=mesh,
                     scratch_types=scratch,
                     compiler_params=pltpu.CompilerParams(
                         use_tc_tiling_on_sc=False))


_sc_agg_deg = _make_sc_agg(True)
_sc_agg = _make_sc_agg(False)


def _transform_body(h_ref, wr_ref, ws_ref, t_ref, s_ref):
    hb = h_ref[...]
    hb16 = hb.astype(jnp.bfloat16)
    for r in range(_R):
        t_ref[r] = jnp.dot(hb16, wr_ref[r], preferred_element_type=jnp.float32)
    s_ref[...] = jnp.dot(hb, ws_ref[...], preferred_element_type=jnp.float32)


def _transform(h, wrel, wself):
    return pl.pallas_call(
        _transform_body,
        grid=(_NBLK,),
        in_specs=[pl.BlockSpec((_BLK, _D), lambda i: (i, 0)),
                  pl.BlockSpec((_R, _D, _D), lambda i: (0, 0, 0)),
                  pl.BlockSpec((_D, _D), lambda i: (0, 0))],
        out_specs=[pl.BlockSpec((_R, _BLK, _D), lambda i: (0, i, 0)),
                   pl.BlockSpec((_BLK, _D), lambda i: (i, 0))],
        out_shape=[jax.ShapeDtypeStruct((_R, _N, _D), jnp.float32),
                   jax.ShapeDtypeStruct((_N, _D), jnp.float32)],
    )(h, wrel, wself)


def _combine_body(a0_ref, a1_ref, d0_ref, d1_ref, s_ref, wr_ref, ws_ref,
                  t_ref, s2_ref, h_ref):
    deg = d0_ref[...][:, 0:1] + d1_ref[...][:, 0:1]
    agg = jnp.concatenate([a0_ref[...], a1_ref[...]], axis=1)
    h1 = jnp.maximum(agg / jnp.maximum(deg, 1.0) + s_ref[...], 0.0)
    h_ref[...] = h1
    s2_ref[...] = jnp.dot(h1, ws_ref[...], preferred_element_type=jnp.float32)
    h16 = h1.astype(jnp.bfloat16)
    for r in range(_R):
        t_ref[r] = jnp.dot(h16, wr_ref[r], preferred_element_type=jnp.float32)


def _combine(a0, a1, d0, d1, s1, wrel, wself):
    return pl.pallas_call(
        _combine_body,
        grid=(_NBLK,),
        in_specs=[pl.BlockSpec((_BLK, _HD), lambda i: (i, 0)),
                  pl.BlockSpec((_BLK, _HD), lambda i: (i, 0)),
                  pl.BlockSpec((_BLK, 8), lambda i: (i, 0)),
                  pl.BlockSpec((_BLK, 8), lambda i: (i, 0)),
                  pl.BlockSpec((_BLK, _D), lambda i: (i, 0)),
                  pl.BlockSpec((_R, _D, _D), lambda i: (0, 0, 0)),
                  pl.BlockSpec((_D, _D), lambda i: (0, 0))],
        out_specs=[pl.BlockSpec((_R, _BLK, _D), lambda i: (0, i, 0)),
                   pl.BlockSpec((_BLK, _D), lambda i: (i, 0)),
                   pl.BlockSpec((_BLK, _D), lambda i: (i, 0))],
        out_shape=[jax.ShapeDtypeStruct((_R, _N, _D), jnp.float32),
                   jax.ShapeDtypeStruct((_N, _D), jnp.float32),
                   jax.ShapeDtypeStruct((_N, _D), jnp.float32)],
    )(a0, a1, d0, d1, s1, wrel, wself)


def _readout_body(a0_ref, a1_ref, d0_ref, d1_ref, s2_ref, h1_ref, gid_ref,
                  hid_ref, tid_ref, rel_ref, spe_ref, rt_ref,
                  wg_ref, wh_ref, wt_ref, wr_ref, ws_ref, bfc_ref, wo_ref,
                  bo_ref, out_ref, G, HD, TL, CNT):
    i = pl.program_id(0)

    @pl.when(i == 0)
    def _():
        G[...] = jnp.zeros_like(G)
        HD[...] = jnp.zeros_like(HD)
        TL[...] = jnp.zeros_like(TL)
        CNT[...] = jnp.zeros_like(CNT)

    deg = d0_ref[...][:, 0:1] + d1_ref[...][:, 0:1]
    agg = jnp.concatenate([a0_ref[...], a1_ref[...]], axis=1)
    h2 = jnp.maximum(agg / jnp.maximum(deg, 1.0) + s2_ref[...], 0.0)
    flat = jnp.concatenate([h1_ref[...], h2], axis=1)

    gb = gid_ref[0, 0, :]
    og = (gb[None, :] == lax.broadcasted_iota(jnp.int32, (_B, _BLK), 0)
          ).astype(jnp.float32)
    G[...] += jnp.dot(og, flat, preferred_element_type=jnp.float32)
    CNT[...] = CNT[...] + jnp.sum(og, axis=1, keepdims=True)

    rowid = lax.broadcasted_iota(jnp.int32, (_B, _BLK), 1) + i * _BLK
    oh = (hid_ref[0, :][:, None] == rowid).astype(jnp.float32)
    ot = (tid_ref[0, :][:, None] == rowid).astype(jnp.float32)
    HD[...] += jnp.dot(oh, flat, preferred_element_type=jnp.float32)
    TL[...] += jnp.dot(ot, flat, preferred_element_type=jnp.float32)

    @pl.when(i == _NBLK - 1)
    def _():
        g_out = G[...] / jnp.maximum(CNT[...][:, 0:1], 1.0)
        orl = (rel_ref[0, :][:, None] ==
               lax.broadcasted_iota(jnp.int32, (_B, _R), 1)).astype(jnp.float32)
        relemb = jnp.dot(orl, rt_ref[...], preferred_element_type=jnp.float32)
        hid = (jnp.dot(g_out, wg_ref[...], preferred_element_type=jnp.float32)
               + jnp.dot(HD[...], wh_ref[...], preferred_element_type=jnp.float32)
               + jnp.dot(TL[...], wt_ref[...], preferred_element_type=jnp.float32)
               + jnp.dot(relemb, wr_ref[...], preferred_element_type=jnp.float32)
               + jnp.dot(spe_ref[...], ws_ref[...], preferred_element_type=jnp.float32)
               + bfc_ref[...])
        hid = jnp.maximum(hid, 0.0)
        res = jnp.sum(hid * wo_ref[...], axis=1, keepdims=True) + bo_ref[0, 0]
        out_ref[...] = jnp.broadcast_to(res, (_B, _D))


def _readout(a0, a1, d0, d1, s2, h1, gid3, hid2, tid2, rel2, spe, rtab,
             wg, wh, wt, wr, ws, bfc2, wo2, bo2):
    def cst(*dims):
        return pl.BlockSpec(dims, lambda i: tuple(0 for _ in dims))
    return pl.pallas_call(
        _readout_body,
        grid=(_NBLK,),
        in_specs=[pl.BlockSpec((_BLK, _HD), lambda i: (i, 0)),
                  pl.BlockSpec((_BLK, _HD), lambda i: (i, 0)),
                  pl.BlockSpec((_BLK, 8), lambda i: (i, 0)),
                  pl.BlockSpec((_BLK, 8), lambda i: (i, 0)),
                  pl.BlockSpec((_BLK, _D), lambda i: (i, 0)),
                  pl.BlockSpec((_BLK, _D), lambda i: (i, 0)),
                  pl.BlockSpec((1, 1, _BLK), lambda i: (i, 0, 0)),
                  cst(1, _B), cst(1, _B), cst(1, _B),
                  cst(_B, 16), cst(_R, _R),
                  cst(2 * _D, _H), cst(2 * _D, _H), cst(2 * _D, _H),
                  cst(_R, _H), cst(16, _H), cst(1, _H), cst(1, _H),
                  cst(1, 1)],
        out_specs=pl.BlockSpec((_B, _D), lambda i: (0, 0)),
        out_shape=jax.ShapeDtypeStruct((_B, _D), jnp.float32),
        scratch_shapes=[pltpu.VMEM((_B, 2 * _D), jnp.float32),
                        pltpu.VMEM((_B, 2 * _D), jnp.float32),
                        pltpu.VMEM((_B, 2 * _D), jnp.float32),
                        pltpu.VMEM((_B, _D), jnp.float32)],
    )(a0, a1, d0, d1, s2, h1, gid3, hid2, tid2, rel2, spe, rtab,
      wg, wh, wt, wr, ws, bfc2, wo2, bo2)


def kernel(x, edge_index, edge_type, graph_ids, head_ids, tail_ids,
           rel_labels, spe, W_rel1, W_self1, W_rel2, W_self2, rel_table,
           W_fc, b_fc, W_out, b_out):
    src3 = edge_index[0].astype(jnp.int32).reshape(_NS, _NCH, _CHUNK)
    dst3 = edge_index[1].astype(jnp.int32).reshape(_NS, _NCH, _CHUNK)
    ety3 = edge_type.astype(jnp.int32).reshape(_NS, _NCH, _CHUNK)

    T1, S1 = _transform(x, W_rel1.astype(jnp.bfloat16), W_self1)
    a0, a1, d0, d1 = _sc_agg_deg(T1.reshape(_N * _R * 2, _HD),
                                 src3, ety3, dst3,
                                 jnp.ones((_CHUNK, 8), jnp.float32),
                                 jnp.zeros((128, 8), jnp.float32))
    T2, S2, H1 = _combine(a0, a1, d0, d1, S1,
                          W_rel2.astype(jnp.bfloat16), W_self2)
    b0, b1 = _sc_agg(T2.reshape(_N * _R * 2, _HD), src3, ety3, dst3)

    OUT = _readout(b0, b1, d0, d1, S2, H1,
                   graph_ids.astype(jnp.int32).reshape(_NBLK, 1, _BLK),
                   head_ids.astype(jnp.int32).reshape(1, _B),
                   tail_ids.astype(jnp.int32).reshape(1, _B),
                   rel_labels.astype(jnp.int32).reshape(1, _B),
                   spe, rel_table,
                   W_fc[0:2 * _D], W_fc[2 * _D:4 * _D], W_fc[4 * _D:6 * _D],
                   W_fc[6 * _D:6 * _D + _R], W_fc[6 * _D + _R:],
                   b_fc.reshape(1, _H), W_out.reshape(1, _H),
                   b_out.reshape(1, 1))
    return OUT[:, :1]


# speedup vs baseline: 4.7714x; 1.1150x over previous
import functools

import jax
import jax.numpy as jnp
from jax import lax
from jax.experimental import pallas as pl
from jax.experimental.pallas import tpu as pltpu
from jax.experimental.pallas import tpu_sc as plsc

_N = 10000
_E = 320000
_D = 128
_R = 32
_B = 128
_H = 16

_NC = 2
_NS = 16
_HD = _D // 2
_EPT = _E // _NS
_CHUNK = 112
_NCHF = _EPT // _CHUNK
_TAIL = _EPT - _NCHF * _CHUNK
_RPT = 632
_RPTL = _N - 15 * _RPT

_BLK = 400
_NBLK = _N // _BLK


def _sc_agg_body(with_deg, *refs):
    if with_deg:
        (t_hbm, src_hbm, ety_hbm, dst_hbm, ones8_hbm, zer8_hbm,
         oa0, oa1, od0, od1,
         src_v, ety_v, dst_v, rowa_v, rowb_v, z_v, agg_sh, sema, semb,
         ones_v, zd_v, deg_sh) = refs
    else:
        (t_hbm, src_hbm, ety_hbm, dst_hbm,
         oa0, oa1,
         src_v, ety_v, dst_v, rowa_v, rowb_v, z_v, agg_sh, sema, semb) = refs

    cid = lax.axis_index("c")
    sid = lax.axis_index("s")

    pltpu.sync_copy(src_hbm.at[sid], src_v)
    pltpu.sync_copy(ety_hbm.at[sid], ety_v)
    pltpu.sync_copy(dst_hbm.at[sid], dst_v)

    def idx_body(r, carry):
        sl = pl.ds(r * 16, 16)
        ety_v[sl] = ety_v[sl] * (2 * _N) + src_v[sl] * 2 + cid
        return carry
    lax.fori_loop(0, _EPT // 16, idx_body, 0)

    def z_body(r, carry):
        for j in range(_HD // 16):
            z_v[r, pl.ds(j * 16, 16)] = jnp.zeros((16,), jnp.float32)
        return carry
    lax.fori_loop(0, 128, z_body, 0)
    base = sid * _RPT

    @pl.when(sid < 15)
    def _():
        for k in range(4):
            pltpu.sync_copy(z_v, agg_sh.at[pl.ds(base + k * 128, 128)])
        pltpu.sync_copy(z_v.at[pl.ds(0, _RPT - 512)],
                        agg_sh.at[pl.ds(base + 512, _RPT - 512)])

    @pl.when(sid == 15)
    def _():
        for k in range(4):
            pltpu.sync_copy(z_v, agg_sh.at[pl.ds(base + k * 128, 128)])
        pltpu.sync_copy(z_v.at[pl.ds(0, _RPTL - 512)],
                        agg_sh.at[pl.ds(base + 512, _RPTL - 512)])

    if with_deg:
        pltpu.sync_copy(ones8_hbm, ones_v)
        pltpu.sync_copy(zer8_hbm, zd_v)

        @pl.when(sid < 15)
        def _():
            for k in range(4):
                pltpu.sync_copy(zd_v, deg_sh.at[pl.ds(base + k * 128, 128)])
            pltpu.sync_copy(zd_v.at[pl.ds(0, _RPT - 512)],
                            deg_sh.at[pl.ds(base + 512, _RPT - 512)])

        @pl.when(sid == 15)
        def _():
            for k in range(4):
                pltpu.sync_copy(zd_v, deg_sh.at[pl.ds(base + k * 128, 128)])
            pltpu.sync_copy(zd_v.at[pl.ds(0, _RPTL - 512)],
                            deg_sh.at[pl.ds(base + 512, _RPTL - 512)])

    plsc.subcore_barrier()

    def gidx(c):
        return ety_v.at[pl.ds(c * _CHUNK, _CHUNK)]

    def scat(row_v, c):
        didx = dst_v.at[pl.ds(c * _CHUNK, _CHUNK)]
        pltpu.sync_copy(row_v, agg_sh.at[didx], add=True)
        if with_deg:
            @pl.when(cid == (c >= _NCHF // 2).astype(jnp.int32))
            def _():
                pltpu.sync_copy(ones_v, deg_sh.at[didx], add=True)

    pltpu.async_copy(t_hbm.at[gidx(0)], rowa_v, sema)

    def pair_body(p, carry):
        c0 = 2 * p
        pltpu.make_async_copy(t_hbm.at[gidx(c0)], rowa_v, sema).wait()
        pltpu.async_copy(t_hbm.at[gidx(c0 + 1)], rowb_v, semb)
        scat(rowa_v, c0)
        pltpu.make_async_copy(t_hbm.at[gidx(c0 + 1)], rowb_v, semb).wait()

        @pl.when(c0 + 2 < _NCHF)
        def _():
            pltpu.async_copy(t_hbm.at[gidx(c0 + 2)], rowa_v, sema)

        scat(rowb_v, c0 + 1)
        return carry
    lax.fori_loop(0, _NCHF // 2, pair_body, 0)

    tgi = ety_v.at[pl.ds(_NCHF * _CHUNK, _TAIL)]
    tdi = dst_v.at[pl.ds(_NCHF * _CHUNK, _TAIL)]
    pltpu.async_copy(t_hbm.at[tgi], rowa_v.at[pl.ds(0, _TAIL)], sema).wait()
    pltpu.sync_copy(rowa_v.at[pl.ds(0, _TAIL)], agg_sh.at[tdi], add=True)
    if with_deg:
        @pl.when(cid == 1)
        def _():
            pltpu.sync_copy(ones_v.at[pl.ds(0, _TAIL)], deg_sh.at[tdi],
                            add=True)

    plsc.subcore_barrier()

    rs = pl.ds(base, _RPT)
    rsl = pl.ds(base, _RPTL)

    @pl.when(jnp.logical_and(cid == 0, sid < 15))
    def _():
        pltpu.sync_copy(agg_sh.at[rs], oa0.at[rs])
        if with_deg:
            pltpu.sync_copy(deg_sh.at[rs], od0.at[rs])

    @pl.when(jnp.logical_and(cid == 0, sid == 15))
    def _():
        pltpu.sync_copy(agg_sh.at[rsl], oa0.at[rsl])
        if with_deg:
            pltpu.sync_copy(deg_sh.at[rsl], od0.at[rsl])

    @pl.when(jnp.logical_and(cid == 1, sid < 15))
    def _():
        pltpu.sync_copy(agg_sh.at[rs], oa1.at[rs])
        if with_deg:
            pltpu.sync_copy(deg_sh.at[rs], od1.at[rs])

    @pl.when(jnp.logical_and(cid == 1, sid == 15))
    def _():
        pltpu.sync_copy(agg_sh.at[rsl], oa1.at[rsl])
        if with_deg:
            pltpu.sync_copy(deg_sh.at[rsl], od1.at[rsl])


def _make_sc_agg(with_deg):
    out_type = [jax.ShapeDtypeStruct((_N, _HD), jnp.float32),
                jax.ShapeDtypeStruct((_N, _HD), jnp.float32)]
    scratch = [
        pltpu.VMEM((_EPT,), jnp.int32),
        pltpu.VMEM((_EPT,), jnp.int32),
        pltpu.VMEM((_EPT,), jnp.int32),
        pltpu.VMEM((_CHUNK, _HD), jnp.float32),
        pltpu.VMEM((_CHUNK, _HD), jnp.float32),
        pltpu.VMEM((128, _HD), jnp.float32),
        pltpu.VMEM_SHARED((_N, _HD), jnp.float32),
        pltpu.SemaphoreType.DMA,
        pltpu.SemaphoreType.DMA,
    ]
    if with_deg:
        out_type += [jax.ShapeDtypeStruct((_N, 8), jnp.float32),
                     jax.ShapeDtypeStruct((_N, 8), jnp.float32)]
        scratch += [
            pltpu.VMEM((_CHUNK, 8), jnp.float32),
            pltpu.VMEM((128, 8), jnp.float32),
            pltpu.VMEM_SHARED((_N, 8), jnp.float32),
        ]
    mesh = plsc.VectorSubcoreMesh(core_axis_name="c", subcore_axis_name="s")
    return pl.kernel(functools.partial(_sc_agg_body, with_deg),
                     out_type=tuple(out_type), mesh=mesh,
                     scratch_types=scratch,
                     compiler_params=pltpu.CompilerParams(
                         use_tc_tiling_on_sc=False))


_sc_agg_deg = _make_sc_agg(True)
_sc_agg = _make_sc_agg(False)


def _transform_body(h_ref, wr_ref, ws_ref, t_ref, s_ref):
    hb = h_ref[...]
    hb16 = hb.astype(jnp.bfloat16)
    for r in range(_R):
        t_ref[r] = jnp.dot(hb16, wr_ref[r], preferred_element_type=jnp.float32)
    s_ref[...] = jnp.dot(hb, ws_ref[...], preferred_element_type=jnp.float32)


def _transform(h, wrel, wself):
    return pl.pallas_call(
        _transform_body,
        grid=(_NBLK,),
        in_specs=[pl.BlockSpec((_BLK, _D), lambda i: (i, 0)),
                  pl.BlockSpec((_R, _D, _D), lambda i: (0, 0, 0)),
                  pl.BlockSpec((_D, _D), lambda i: (0, 0))],
        out_specs=[pl.BlockSpec((_R, _BLK, _D), lambda i: (0, i, 0)),
                   pl.BlockSpec((_BLK, _D), lambda i: (i, 0))],
        out_shape=[jax.ShapeDtypeStruct((_R, _N, _D), jnp.float32),
                   jax.ShapeDtypeStruct((_N, _D), jnp.float32)],
    )(h, wrel, wself)


def _combine_body(a0_ref, a1_ref, d0_ref, d1_ref, s_ref, wr_ref, ws_ref,
                  t_ref, s2_ref, h_ref):
    deg = d0_ref[...][:, 0:1] + d1_ref[...][:, 0:1]
    agg = jnp.concatenate([a0_ref[...], a1_ref[...]], axis=1)
    h1 = jnp.maximum(agg / jnp.maximum(deg, 1.0) + s_ref[...], 0.0)
    h_ref[...] = h1
    s2_ref[...] = jnp.dot(h1, ws_ref[...], preferred_element_type=jnp.float32)
    h16 = h1.astype(jnp.bfloat16)
    for r in range(_R):
        t_ref[r] = jnp.dot(h16, wr_ref[r], preferred_element_type=jnp.float32)


def _combine(a0, a1, d0, d1, s1, wrel, wself):
    return pl.pallas_call(
        _combine_body,
        grid=(_NBLK,),
        in_specs=[pl.BlockSpec((_BLK, _HD), lambda i: (i, 0)),
                  pl.BlockSpec((_BLK, _HD), lambda i: (i, 0)),
                  pl.BlockSpec((_BLK, 8), lambda i: (i, 0)),
                  pl.BlockSpec((_BLK, 8), lambda i: (i, 0)),
                  pl.BlockSpec((_BLK, _D), lambda i: (i, 0)),
                  pl.BlockSpec((_R, _D, _D), lambda i: (0, 0, 0)),
                  pl.BlockSpec((_D, _D), lambda i: (0, 0))],
        out_specs=[pl.BlockSpec((_R, _BLK, _D), lambda i: (0, i, 0)),
                   pl.BlockSpec((_BLK, _D), lambda i: (i, 0)),
                   pl.BlockSpec((_BLK, _D), lambda i: (i, 0))],
        out_shape=[jax.ShapeDtypeStruct((_R, _N, _D), jnp.float32),
                   jax.ShapeDtypeStruct((_N, _D), jnp.float32),
                   jax.ShapeDtypeStruct((_N, _D), jnp.float32)],
    )(a0, a1, d0, d1, s1, wrel, wself)


def _readout_body(a0_ref, a1_ref, d0_ref, d1_ref, s2_ref, h1_ref, gid_ref,
                  hid_ref, tid_ref, rel_ref, spe_ref, rt_ref,
                  wg_ref, wh_ref, wt_ref, wr_ref, ws_ref, bfc_ref, wo_ref,
                  bo_ref, out_ref, G, HD, TL, CNT):
    i = pl.program_id(0)

    @pl.when(i == 0)
    def _():
        G[...] = jnp.zeros_like(G)
        HD[...] = jnp.zeros_like(HD)
        TL[...] = jnp.zeros_like(TL)
        CNT[...] = jnp.zeros_like(CNT)

    deg = d0_ref[...][:, 0:1] + d1_ref[...][:, 0:1]
    agg = jnp.concatenate([a0_ref[...], a1_ref[...]], axis=1)
    h2 = jnp.maximum(agg / jnp.maximum(deg, 1.0) + s2_ref[...], 0.0)
    flat = jnp.concatenate([h1_ref[...], h2], axis=1)

    gb = gid_ref[0, 0, :]
    og = (gb[None, :] == lax.broadcasted_iota(jnp.int32, (_B, _BLK), 0)
          ).astype(jnp.float32)
    G[...] += jnp.dot(og, flat, preferred_element_type=jnp.float32)
    CNT[...] = CNT[...] + jnp.sum(og, axis=1, keepdims=True)

    rowid = lax.broadcasted_iota(jnp.int32, (_B, _BLK), 1) + i * _BLK
    oh = (hid_ref[0, :][:, None] == rowid).astype(jnp.float32)
    ot = (tid_ref[0, :][:, None] == rowid).astype(jnp.float32)
    HD[...] += jnp.dot(oh, flat, preferred_element_type=jnp.float32)
    TL[...] += jnp.dot(ot, flat, preferred_element_type=jnp.float32)

    @pl.when(i == _NBLK - 1)
    def _():
        g_out = G[...] / jnp.maximum(CNT[...][:, 0:1], 1.0)
        orl = (rel_ref[0, :][:, None] ==
               lax.broadcasted_iota(jnp.int32, (_B, _R), 1)).astype(jnp.float32)
        relemb = jnp.dot(orl, rt_ref[...], preferred_element_type=jnp.float32)
        hid = (jnp.dot(g_out, wg_ref[...], preferred_element_type=jnp.float32)
               + jnp.dot(HD[...], wh_ref[...], preferred_element_type=jnp.float32)
               + jnp.dot(TL[...], wt_ref[...], preferred_element_type=jnp.float32)
               + jnp.dot(relemb, wr_ref[...], preferred_element_type=jnp.float32)
               + jnp.dot(spe_ref[...], ws_ref[...], preferred_element_type=jnp.float32)
               + bfc_ref[...])
        hid = jnp.maximum(hid, 0.0)
        res = jnp.sum(hid * wo_ref[...], axis=1, keepdims=True) + bo_ref[0, 0]
        out_ref[...] = jnp.broadcast_to(res, (_B, _D))


def _readout(a0, a1, d0, d1, s2, h1, gid3, hid2, tid2, rel2, spe, rtab,
             wg, wh, wt, wr, ws, bfc2, wo2, bo2):
    def cst(*dims):
        return pl.BlockSpec(dims, lambda i: tuple(0 for _ in dims))
    return pl.pallas_call(
        _readout_body,
        grid=(_NBLK,),
        in_specs=[pl.BlockSpec((_BLK, _HD), lambda i: (i, 0)),
                  pl.BlockSpec((_BLK, _HD), lambda i: (i, 0)),
                  pl.BlockSpec((_BLK, 8), lambda i: (i, 0)),
                  pl.BlockSpec((_BLK, 8), lambda i: (i, 0)),
                  pl.BlockSpec((_BLK, _D), lambda i: (i, 0)),
                  pl.BlockSpec((_BLK, _D), lambda i: (i, 0)),
                  pl.BlockSpec((1, 1, _BLK), lambda i: (i, 0, 0)),
                  cst(1, _B), cst(1, _B), cst(1, _B),
                  cst(_B, 16), cst(_R, _R),
                  cst(2 * _D, _H), cst(2 * _D, _H), cst(2 * _D, _H),
                  cst(_R, _H), cst(16, _H), cst(1, _H), cst(1, _H),
                  cst(1, 1)],
        out_specs=pl.BlockSpec((_B, _D), lambda i: (0, 0)),
        out_shape=jax.ShapeDtypeStruct((_B, _D), jnp.float32),
        scratch_shapes=[pltpu.VMEM((_B, 2 * _D), jnp.float32),
                        pltpu.VMEM((_B, 2 * _D), jnp.float32),
                        pltpu.VMEM((_B, 2 * _D), jnp.float32),
                        pltpu.VMEM((_B, _D), jnp.float32)],
    )(a0, a1, d0, d1, s2, h1, gid3, hid2, tid2, rel2, spe, rtab,
      wg, wh, wt, wr, ws, bfc2, wo2, bo2)


def kernel(x, edge_index, edge_type, graph_ids, head_ids, tail_ids,
           rel_labels, spe, W_rel1, W_self1, W_rel2, W_self2, rel_table,
           W_fc, b_fc, W_out, b_out):
    src3 = edge_index[0].astype(jnp.int32).reshape(_NS, _EPT)
    dst3 = edge_index[1].astype(jnp.int32).reshape(_NS, _EPT)
    ety3 = edge_type.astype(jnp.int32).reshape(_NS, _EPT)

    T1, S1 = _transform(x, W_rel1.astype(jnp.bfloat16), W_self1)
    a0, a1, d0, d1 = _sc_agg_deg(T1.reshape(_N * _R * 2, _HD),
                                 src3, ety3, dst3,
                                 jnp.ones((_CHUNK, 8), jnp.float32),
                                 jnp.zeros((128, 8), jnp.float32))
    T2, S2, H1 = _combine(a0, a1, d0, d1, S1,
                          W_rel2.astype(jnp.bfloat16), W_self2)
    b0, b1 = _sc_agg(T2.reshape(_N * _R * 2, _HD), src3, ety3, dst3)

    OUT = _readout(b0, b1, d0, d1, S2, H1,
                   graph_ids.astype(jnp.int32).reshape(_NBLK, 1, _BLK),
                   head_ids.astype(jnp.int32).reshape(1, _B),
                   tail_ids.astype(jnp.int32).reshape(1, _B),
                   rel_labels.astype(jnp.int32).reshape(1, _B),
                   spe, rel_table,
                   W_fc[0:2 * _D], W_fc[2 * _D:4 * _D], W_fc[4 * _D:6 * _D],
                   W_fc[6 * _D:6 * _D + _R], W_fc[6 * _D + _R:],
                   b_fc.reshape(1, _H), W_out.reshape(1, _H),
                   b_out.reshape(1, 1))
    return OUT[:, :1]


# speedup vs baseline: 4.9032x; 1.0276x over previous
import functools

import jax
import jax.numpy as jnp
from jax import lax
from jax.experimental import pallas as pl
from jax.experimental.pallas import tpu as pltpu
from jax.experimental.pallas import tpu_sc as plsc

_N = 10000
_E = 320000
_D = 128
_R = 32
_B = 128
_H = 16

_NC = 2
_NS = 16
_HD = _D // 2
_EPT = _E // _NS
_CHUNK = 112
_NCHF = _EPT // _CHUNK
_TAIL = _EPT - _NCHF * _CHUNK
_RPT = 632
_RPTL = _N - 15 * _RPT

_BLK = 1000
_NBLK = _N // _BLK


def _sc_agg_body(with_deg, *refs):
    if with_deg:
        (t_hbm, src_hbm, ety_hbm, dst_hbm, ones8_hbm, zer8_hbm,
         oa0, oa1, od0, od1,
         src_v, ety_v, dst_v, rowa_v, rowb_v, z_v, agg_sh, sema, semb,
         ones_v, zd_v, deg_sh) = refs
    else:
        (t_hbm, src_hbm, ety_hbm, dst_hbm,
         oa0, oa1,
         src_v, ety_v, dst_v, rowa_v, rowb_v, z_v, agg_sh, sema, semb) = refs

    cid = lax.axis_index("c")
    sid = lax.axis_index("s")

    pltpu.sync_copy(src_hbm.at[sid], src_v)
    pltpu.sync_copy(ety_hbm.at[sid], ety_v)
    pltpu.sync_copy(dst_hbm.at[sid], dst_v)

    def idx_body(r, carry):
        sl = pl.ds(r * 16, 16)
        ety_v[sl] = ety_v[sl] * (2 * _N) + src_v[sl] * 2 + cid
        return carry
    lax.fori_loop(0, _EPT // 16, idx_body, 0)

    def z_body(r, carry):
        for j in range(_HD // 16):
            z_v[r, pl.ds(j * 16, 16)] = jnp.zeros((16,), jnp.float32)
        return carry
    lax.fori_loop(0, 128, z_body, 0)
    base = sid * _RPT

    @pl.when(sid < 15)
    def _():
        for k in range(4):
            pltpu.sync_copy(z_v, agg_sh.at[pl.ds(base + k * 128, 128)])
        pltpu.sync_copy(z_v.at[pl.ds(0, _RPT - 512)],
                        agg_sh.at[pl.ds(base + 512, _RPT - 512)])

    @pl.when(sid == 15)
    def _():
        for k in range(4):
            pltpu.sync_copy(z_v, agg_sh.at[pl.ds(base + k * 128, 128)])
        pltpu.sync_copy(z_v.at[pl.ds(0, _RPTL - 512)],
                        agg_sh.at[pl.ds(base + 512, _RPTL - 512)])

    if with_deg:
        pltpu.sync_copy(ones8_hbm, ones_v)
        pltpu.sync_copy(zer8_hbm, zd_v)

        @pl.when(sid < 15)
        def _():
            for k in range(4):
                pltpu.sync_copy(zd_v, deg_sh.at[pl.ds(base + k * 128, 128)])
            pltpu.sync_copy(zd_v.at[pl.ds(0, _RPT - 512)],
                            deg_sh.at[pl.ds(base + 512, _RPT - 512)])

        @pl.when(sid == 15)
        def _():
            for k in range(4):
                pltpu.sync_copy(zd_v, deg_sh.at[pl.ds(base + k * 128, 128)])
            pltpu.sync_copy(zd_v.at[pl.ds(0, _RPTL - 512)],
                            deg_sh.at[pl.ds(base + 512, _RPTL - 512)])

    plsc.subcore_barrier()

    def gidx(c):
        return ety_v.at[pl.ds(c * _CHUNK, _CHUNK)]

    def scat(row_v, c):
        didx = dst_v.at[pl.ds(c * _CHUNK, _CHUNK)]
        pltpu.sync_copy(row_v, agg_sh.at[didx], add=True)
        if with_deg:
            @pl.when(cid == (c >= _NCHF // 2).astype(jnp.int32))
            def _():
                pltpu.sync_copy(ones_v, deg_sh.at[didx], add=True)

    pltpu.async_copy(t_hbm.at[gidx(0)], rowa_v, sema)

    def pair_body(p, carry):
        c0 = 2 * p
        pltpu.make_async_copy(t_hbm.at[gidx(c0)], rowa_v, sema).wait()
        pltpu.async_copy(t_hbm.at[gidx(c0 + 1)], rowb_v, semb)
        scat(rowa_v, c0)
        pltpu.make_async_copy(t_hbm.at[gidx(c0 + 1)], rowb_v, semb).wait()

        @pl.when(c0 + 2 < _NCHF)
        def _():
            pltpu.async_copy(t_hbm.at[gidx(c0 + 2)], rowa_v, sema)

        scat(rowb_v, c0 + 1)
        return carry
    lax.fori_loop(0, _NCHF // 2, pair_body, 0)

    tgi = ety_v.at[pl.ds(_NCHF * _CHUNK, _TAIL)]
    tdi = dst_v.at[pl.ds(_NCHF * _CHUNK, _TAIL)]
    pltpu.async_copy(t_hbm.at[tgi], rowa_v.at[pl.ds(0, _TAIL)], sema).wait()
    pltpu.sync_copy(rowa_v.at[pl.ds(0, _TAIL)], agg_sh.at[tdi], add=True)
    if with_deg:
        @pl.when(cid == 1)
        def _():
            pltpu.sync_copy(ones_v.at[pl.ds(0, _TAIL)], deg_sh.at[tdi],
                            add=True)

    plsc.subcore_barrier()

    rs = pl.ds(base, _RPT)
    rsl = pl.ds(base, _RPTL)

    @pl.when(jnp.logical_and(cid == 0, sid < 15))
    def _():
        pltpu.sync_copy(agg_sh.at[rs], oa0.at[rs])
        if with_deg:
            pltpu.sync_copy(deg_sh.at[rs], od0.at[rs])

    @pl.when(jnp.logical_and(cid == 0, sid == 15))
    def _():
        pltpu.sync_copy(agg_sh.at[rsl], oa0.at[rsl])
        if with_deg:
            pltpu.sync_copy(deg_sh.at[rsl], od0.at[rsl])

    @pl.when(jnp.logical_and(cid == 1, sid < 15))
    def _():
        pltpu.sync_copy(agg_sh.at[rs], oa1.at[rs])
        if with_deg:
            pltpu.sync_copy(deg_sh.at[rs], od1.at[rs])

    @pl.when(jnp.logical_and(cid == 1, sid == 15))
    def _():
        pltpu.sync_copy(agg_sh.at[rsl], oa1.at[rsl])
        if with_deg:
            pltpu.sync_copy(deg_sh.at[rsl], od1.at[rsl])


def _make_sc_agg(with_deg):
    out_type = [jax.ShapeDtypeStruct((_N, _HD), jnp.float32),
                jax.ShapeDtypeStruct((_N, _HD), jnp.float32)]
    scratch = [
        pltpu.VMEM((_EPT,), jnp.int32),
        pltpu.VMEM((_EPT,), jnp.int32),
        pltpu.VMEM((_EPT,), jnp.int32),
        pltpu.VMEM((_CHUNK, _HD), jnp.float32),
        pltpu.VMEM((_CHUNK, _HD), jnp.float32),
        pltpu.VMEM((128, _HD), jnp.float32),
        pltpu.VMEM_SHARED((_N, _HD), jnp.float32),
        pltpu.SemaphoreType.DMA,
        pltpu.SemaphoreType.DMA,
    ]
    if with_deg:
        out_type += [jax.ShapeDtypeStruct((_N, 8), jnp.float32),
                     jax.ShapeDtypeStruct((_N, 8), jnp.float32)]
        scratch += [
            pltpu.VMEM((_CHUNK, 8), jnp.float32),
            pltpu.VMEM((128, 8), jnp.float32),
            pltpu.VMEM_SHARED((_N, 8), jnp.float32),
        ]
    mesh = plsc.VectorSubcoreMesh(core_axis_name="c", subcore_axis_name="s")
    return pl.kernel(functools.partial(_sc_agg_body, with_deg),
                     out_type=tuple(out_type), mesh=mesh,
                     scratch_types=scratch,
                     compiler_params=pltpu.CompilerParams(
                         use_tc_tiling_on_sc=False))


_sc_agg_deg = _make_sc_agg(True)
_sc_agg = _make_sc_agg(False)


def _transform_body(h_ref, wr_ref, ws_ref, t_ref, s_ref):
    hb = h_ref[...]
    hb16 = hb.astype(jnp.bfloat16)
    for r in range(_R):
        t_ref[r] = jnp.dot(hb16, wr_ref[r], preferred_element_type=jnp.float32)
    s_ref[...] = jnp.dot(hb, ws_ref[...], preferred_element_type=jnp.float32)


def _transform(h, wrel, wself):
    return pl.pallas_call(
        _transform_body,
        grid=(_NBLK,),
        in_specs=[pl.BlockSpec((_BLK, _D), lambda i: (i, 0)),
                  pl.BlockSpec((_R, _D, _D), lambda i: (0, 0, 0)),
                  pl.BlockSpec((_D, _D), lambda i: (0, 0))],
        out_specs=[pl.BlockSpec((_R, _BLK, _D), lambda i: (0, i, 0)),
                   pl.BlockSpec((_BLK, _D), lambda i: (i, 0))],
        out_shape=[jax.ShapeDtypeStruct((_R, _N, _D), jnp.float32),
                   jax.ShapeDtypeStruct((_N, _D), jnp.float32)],
    )(h, wrel, wself)


def _combine_body(a0_ref, a1_ref, d0_ref, d1_ref, s_ref, wr_ref, ws_ref,
                  t_ref, s2_ref, h_ref):
    deg = d0_ref[...][:, 0:1] + d1_ref[...][:, 0:1]
    agg = jnp.concatenate([a0_ref[...], a1_ref[...]], axis=1)
    h1 = jnp.maximum(agg / jnp.maximum(deg, 1.0) + s_ref[...], 0.0)
    h_ref[...] = h1
    s2_ref[...] = jnp.dot(h1, ws_ref[...], preferred_element_type=jnp.float32)
    h16 = h1.astype(jnp.bfloat16)
    for r in range(_R):
        t_ref[r] = jnp.dot(h16, wr_ref[r], preferred_element_type=jnp.float32)


def _combine(a0, a1, d0, d1, s1, wrel, wself):
    return pl.pallas_call(
        _combine_body,
        grid=(_NBLK,),
        in_specs=[pl.BlockSpec((_BLK, _HD), lambda i: (i, 0)),
                  pl.BlockSpec((_BLK, _HD), lambda i: (i, 0)),
                  pl.BlockSpec((_BLK, 8), lambda i: (i, 0)),
                  pl.BlockSpec((_BLK, 8), lambda i: (i, 0)),
                  pl.BlockSpec((_BLK, _D), lambda i: (i, 0)),
                  pl.BlockSpec((_R, _D, _D), lambda i: (0, 0, 0)),
                  pl.BlockSpec((_D, _D), lambda i: (0, 0))],
        out_specs=[pl.BlockSpec((_R, _BLK, _D), lambda i: (0, i, 0)),
                   pl.BlockSpec((_BLK, _D), lambda i: (i, 0)),
                   pl.BlockSpec((_BLK, _D), lambda i: (i, 0))],
        out_shape=[jax.ShapeDtypeStruct((_R, _N, _D), jnp.float32),
                   jax.ShapeDtypeStruct((_N, _D), jnp.float32),
                   jax.ShapeDtypeStruct((_N, _D), jnp.float32)],
    )(a0, a1, d0, d1, s1, wrel, wself)


def _readout_body(a0_ref, a1_ref, d0_ref, d1_ref, s2_ref, h1_ref, gid_ref,
                  hid_ref, tid_ref, rel_ref, spe_ref, rt_ref,
                  wg_ref, wh_ref, wt_ref, wr_ref, ws_ref, bfc_ref, wo_ref,
                  bo_ref, out_ref, G, HD, TL, CNT):
    i = pl.program_id(0)

    @pl.when(i == 0)
    def _():
        G[...] = jnp.zeros_like(G)
        HD[...] = jnp.zeros_like(HD)
        TL[...] = jnp.zeros_like(TL)
        CNT[...] = jnp.zeros_like(CNT)

    deg = d0_ref[...][:, 0:1] + d1_ref[...][:, 0:1]
    agg = jnp.concatenate([a0_ref[...], a1_ref[...]], axis=1)
    h2 = jnp.maximum(agg / jnp.maximum(deg, 1.0) + s2_ref[...], 0.0)
    flat = jnp.concatenate([h1_ref[...], h2], axis=1)

    gb = gid_ref[0, 0, :]
    og = (gb[None, :] == lax.broadcasted_iota(jnp.int32, (_B, _BLK), 0)
          ).astype(jnp.float32)
    G[...] += jnp.dot(og, flat, preferred_element_type=jnp.float32)
    CNT[...] = CNT[...] + jnp.sum(og, axis=1, keepdims=True)

    rowid = lax.broadcasted_iota(jnp.int32, (_B, _BLK), 1) + i * _BLK
    oh = (hid_ref[0, :][:, None] == rowid).astype(jnp.float32)
    ot = (tid_ref[0, :][:, None] == rowid).astype(jnp.float32)
    HD[...] += jnp.dot(oh, flat, preferred_element_type=jnp.float32)
    TL[...] += jnp.dot(ot, flat, preferred_element_type=jnp.float32)

    @pl.when(i == _NBLK - 1)
    def _():
        g_out = G[...] / jnp.maximum(CNT[...][:, 0:1], 1.0)
        orl = (rel_ref[0, :][:, None] ==
               lax.broadcasted_iota(jnp.int32, (_B, _R), 1)).astype(jnp.float32)
        relemb = jnp.dot(orl, rt_ref[...], preferred_element_type=jnp.float32)
        hid = (jnp.dot(g_out, wg_ref[...], preferred_element_type=jnp.float32)
               + jnp.dot(HD[...], wh_ref[...], preferred_element_type=jnp.float32)
               + jnp.dot(TL[...], wt_ref[...], preferred_element_type=jnp.float32)
               + jnp.dot(relemb, wr_ref[...], preferred_element_type=jnp.float32)
               + jnp.dot(spe_ref[...], ws_ref[...], preferred_element_type=jnp.float32)
               + bfc_ref[...])
        hid = jnp.maximum(hid, 0.0)
        res = jnp.sum(hid * wo_ref[...], axis=1, keepdims=True) + bo_ref[0, 0]
        out_ref[...] = jnp.broadcast_to(res, (_B, _D))


def _readout(a0, a1, d0, d1, s2, h1, gid3, hid2, tid2, rel2, spe, rtab,
             wg, wh, wt, wr, ws, bfc2, wo2, bo2):
    def cst(*dims):
        return pl.BlockSpec(dims, lambda i: tuple(0 for _ in dims))
    return pl.pallas_call(
        _readout_body,
        grid=(_NBLK,),
        in_specs=[pl.BlockSpec((_BLK, _HD), lambda i: (i, 0)),
                  pl.BlockSpec((_BLK, _HD), lambda i: (i, 0)),
                  pl.BlockSpec((_BLK, 8), lambda i: (i, 0)),
                  pl.BlockSpec((_BLK, 8), lambda i: (i, 0)),
                  pl.BlockSpec((_BLK, _D), lambda i: (i, 0)),
                  pl.BlockSpec((_BLK, _D), lambda i: (i, 0)),
                  pl.BlockSpec((1, 1, _BLK), lambda i: (i, 0, 0)),
                  cst(1, _B), cst(1, _B), cst(1, _B),
                  cst(_B, 16), cst(_R, _R),
                  cst(2 * _D, _H), cst(2 * _D, _H), cst(2 * _D, _H),
                  cst(_R, _H), cst(16, _H), cst(1, _H), cst(1, _H),
                  cst(1, 1)],
        out_specs=pl.BlockSpec((_B, _D), lambda i: (0, 0)),
        out_shape=jax.ShapeDtypeStruct((_B, _D), jnp.float32),
        scratch_shapes=[pltpu.VMEM((_B, 2 * _D), jnp.float32),
                        pltpu.VMEM((_B, 2 * _D), jnp.float32),
                        pltpu.VMEM((_B, 2 * _D), jnp.float32),
                        pltpu.VMEM((_B, _D), jnp.float32)],
    )(a0, a1, d0, d1, s2, h1, gid3, hid2, tid2, rel2, spe, rtab,
      wg, wh, wt, wr, ws, bfc2, wo2, bo2)


def kernel(x, edge_index, edge_type, graph_ids, head_ids, tail_ids,
           rel_labels, spe, W_rel1, W_self1, W_rel2, W_self2, rel_table,
           W_fc, b_fc, W_out, b_out):
    src3 = edge_index[0].astype(jnp.int32).reshape(_NS, _EPT)
    dst3 = edge_index[1].astype(jnp.int32).reshape(_NS, _EPT)
    ety3 = edge_type.astype(jnp.int32).reshape(_NS, _EPT)

    T1, S1 = _transform(x, W_rel1.astype(jnp.bfloat16), W_self1)
    a0, a1, d0, d1 = _sc_agg_deg(T1.reshape(_N * _R * 2, _HD),
                                 src3, ety3, dst3,
                                 jnp.ones((_CHUNK, 8), jnp.float32),
                                 jnp.zeros((128, 8), jnp.float32))
    T2, S2, H1 = _combine(a0, a1, d0, d1, S1,
                          W_rel2.astype(jnp.bfloat16), W_self2)
    b0, b1 = _sc_agg(T2.reshape(_N * _R * 2, _HD), src3, ety3, dst3)

    OUT = _readout(b0, b1, d0, d1, S2, H1,
                   graph_ids.astype(jnp.int32).reshape(_NBLK, 1, _BLK),
                   head_ids.astype(jnp.int32).reshape(1, _B),
                   tail_ids.astype(jnp.int32).reshape(1, _B),
                   rel_labels.astype(jnp.int32).reshape(1, _B),
                   spe, rel_table,
                   W_fc[0:2 * _D], W_fc[2 * _D:4 * _D], W_fc[4 * _D:6 * _D],
                   W_fc[6 * _D:6 * _D + _R], W_fc[6 * _D + _R:],
                   b_fc.reshape(1, _H), W_out.reshape(1, _H),
                   b_out.reshape(1, 1))
    return OUT[:, :1]


# speedup vs baseline: 4.9118x; 1.0017x over previous
import functools

import jax
import jax.numpy as jnp
from jax import lax
from jax.experimental import pallas as pl
from jax.experimental.pallas import tpu as pltpu
from jax.experimental.pallas import tpu_sc as plsc

_N = 10000
_E = 320000
_D = 128
_R = 32
_B = 128
_H = 16

_NC = 2
_NS = 16
_HD = _D // 2
_EPT = _E // _NS
_CHUNK = 112
_NCHF = _EPT // _CHUNK
_TAIL = _EPT - _NCHF * _CHUNK
_RPT = 632
_RPTL = _N - 15 * _RPT

_BLK = 1000
_NBLK = _N // _BLK


def _sc_agg_body(with_deg, *refs):
    if with_deg:
        (t_hbm, src_hbm, ety_hbm, dst_hbm, ones8_hbm, zer8_hbm,
         oa0, oa1, od0, od1,
         src_v, ety_v, dst_v, rowa_v, rowb_v, z_v, agg_sh, sema, semb,
         ones_v, zd_v, deg_sh) = refs
    else:
        (t_hbm, src_hbm, ety_hbm, dst_hbm,
         oa0, oa1,
         src_v, ety_v, dst_v, rowa_v, rowb_v, z_v, agg_sh, sema, semb) = refs

    cid = lax.axis_index("c")
    sid = lax.axis_index("s")

    pltpu.sync_copy(src_hbm.at[sid], src_v)
    pltpu.sync_copy(ety_hbm.at[sid], ety_v)
    pltpu.sync_copy(dst_hbm.at[sid], dst_v)

    def idx_body(r, carry):
        sl = pl.ds(r * 16, 16)
        ety_v[sl] = ety_v[sl] * (2 * _N) + src_v[sl] * 2 + cid
        return carry
    lax.fori_loop(0, _EPT // 16, idx_body, 0)

    def z_body(r, carry):
        for j in range(_HD // 16):
            z_v[r, pl.ds(j * 16, 16)] = jnp.zeros((16,), jnp.float32)
        return carry
    lax.fori_loop(0, 128, z_body, 0)
    base = sid * _RPT

    @pl.when(sid < 15)
    def _():
        for k in range(4):
            pltpu.sync_copy(z_v, agg_sh.at[pl.ds(base + k * 128, 128)])
        pltpu.sync_copy(z_v.at[pl.ds(0, _RPT - 512)],
                        agg_sh.at[pl.ds(base + 512, _RPT - 512)])

    @pl.when(sid == 15)
    def _():
        for k in range(4):
            pltpu.sync_copy(z_v, agg_sh.at[pl.ds(base + k * 128, 128)])
        pltpu.sync_copy(z_v.at[pl.ds(0, _RPTL - 512)],
                        agg_sh.at[pl.ds(base + 512, _RPTL - 512)])

    if with_deg:
        pltpu.sync_copy(ones8_hbm, ones_v)
        pltpu.sync_copy(zer8_hbm, zd_v)

        @pl.when(sid < 15)
        def _():
            for k in range(4):
                pltpu.sync_copy(zd_v, deg_sh.at[pl.ds(base + k * 128, 128)])
            pltpu.sync_copy(zd_v.at[pl.ds(0, _RPT - 512)],
                            deg_sh.at[pl.ds(base + 512, _RPT - 512)])

        @pl.when(sid == 15)
        def _():
            for k in range(4):
                pltpu.sync_copy(zd_v, deg_sh.at[pl.ds(base + k * 128, 128)])
            pltpu.sync_copy(zd_v.at[pl.ds(0, _RPTL - 512)],
                            deg_sh.at[pl.ds(base + 512, _RPTL - 512)])

    plsc.subcore_barrier()

    def gidx(c):
        return ety_v.at[pl.ds(c * _CHUNK, _CHUNK)]

    def scat(row_v, c):
        didx = dst_v.at[pl.ds(c * _CHUNK, _CHUNK)]
        pltpu.sync_copy(row_v, agg_sh.at[didx], add=True)
        if with_deg:
            @pl.when(cid == (c >= _NCHF // 2).astype(jnp.int32))
            def _():
                pltpu.sync_copy(ones_v, deg_sh.at[didx], add=True)

    pltpu.async_copy(t_hbm.at[gidx(0)], rowa_v, sema)

    def pair_body(p, carry):
        c0 = 2 * p
        pltpu.make_async_copy(t_hbm.at[gidx(c0)], rowa_v, sema).wait()
        pltpu.async_copy(t_hbm.at[gidx(c0 + 1)], rowb_v, semb)
        scat(rowa_v, c0)
        pltpu.make_async_copy(t_hbm.at[gidx(c0 + 1)], rowb_v, semb).wait()

        @pl.when(c0 + 2 < _NCHF)
        def _():
            pltpu.async_copy(t_hbm.at[gidx(c0 + 2)], rowa_v, sema)

        scat(rowb_v, c0 + 1)
        return carry
    lax.fori_loop(0, _NCHF // 2, pair_body, 0)

    tgi = ety_v.at[pl.ds(_NCHF * _CHUNK, _TAIL)]
    tdi = dst_v.at[pl.ds(_NCHF * _CHUNK, _TAIL)]
    pltpu.async_copy(t_hbm.at[tgi], rowa_v.at[pl.ds(0, _TAIL)], sema).wait()
    pltpu.sync_copy(rowa_v.at[pl.ds(0, _TAIL)], agg_sh.at[tdi], add=True)
    if with_deg:
        @pl.when(cid == 1)
        def _():
            pltpu.sync_copy(ones_v.at[pl.ds(0, _TAIL)], deg_sh.at[tdi],
                            add=True)

    plsc.subcore_barrier()

    rs = pl.ds(base, _RPT)
    rsl = pl.ds(base, _RPTL)

    @pl.when(jnp.logical_and(cid == 0, sid < 15))
    def _():
        pltpu.sync_copy(agg_sh.at[rs], oa0.at[rs])
        if with_deg:
            pltpu.sync_copy(deg_sh.at[rs], od0.at[rs])

    @pl.when(jnp.logical_and(cid == 0, sid == 15))
    def _():
        pltpu.sync_copy(agg_sh.at[rsl], oa0.at[rsl])
        if with_deg:
            pltpu.sync_copy(deg_sh.at[rsl], od0.at[rsl])

    @pl.when(jnp.logical_and(cid == 1, sid < 15))
    def _():
        pltpu.sync_copy(agg_sh.at[rs], oa1.at[rs])
        if with_deg:
            pltpu.sync_copy(deg_sh.at[rs], od1.at[rs])

    @pl.when(jnp.logical_and(cid == 1, sid == 15))
    def _():
        pltpu.sync_copy(agg_sh.at[rsl], oa1.at[rsl])
        if with_deg:
            pltpu.sync_copy(deg_sh.at[rsl], od1.at[rsl])


def _make_sc_agg(with_deg):
    out_type = [jax.ShapeDtypeStruct((_N, _HD), jnp.float32),
                jax.ShapeDtypeStruct((_N, _HD), jnp.float32)]
    scratch = [
        pltpu.VMEM((_EPT,), jnp.int32),
        pltpu.VMEM((_EPT,), jnp.int32),
        pltpu.VMEM((_EPT,), jnp.int32),
        pltpu.VMEM((_CHUNK, _HD), jnp.float32),
        pltpu.VMEM((_CHUNK, _HD), jnp.float32),
        pltpu.VMEM((128, _HD), jnp.float32),
        pltpu.VMEM_SHARED((_N, _HD), jnp.float32),
        pltpu.SemaphoreType.DMA,
        pltpu.SemaphoreType.DMA,
    ]
    if with_deg:
        out_type += [jax.ShapeDtypeStruct((_N, 8), jnp.float32),
                     jax.ShapeDtypeStruct((_N, 8), jnp.float32)]
        scratch += [
            pltpu.VMEM((_CHUNK, 8), jnp.float32),
            pltpu.VMEM((128, 8), jnp.float32),
            pltpu.VMEM_SHARED((_N, 8), jnp.float32),
        ]
    mesh = plsc.VectorSubcoreMesh(core_axis_name="c", subcore_axis_name="s")
    return pl.kernel(functools.partial(_sc_agg_body, with_deg),
                     out_type=tuple(out_type), mesh=mesh,
                     scratch_types=scratch,
                     compiler_params=pltpu.CompilerParams(
                         use_tc_tiling_on_sc=False))


_sc_agg_deg = _make_sc_agg(True)
_sc_agg = _make_sc_agg(False)


def _transform_body(h_ref, wr_ref, ws_ref, t_ref, s_ref):
    hb = h_ref[...]
    hb16 = hb.astype(jnp.bfloat16)
    for r in range(_R):
        t_ref[r] = jnp.dot(hb16, wr_ref[r], preferred_element_type=jnp.float32)
    s_ref[...] = jnp.dot(hb, ws_ref[...], preferred_element_type=jnp.float32)


def _transform(h, wrel, wself):
    return pl.pallas_call(
        _transform_body,
        grid=(_NBLK,),
        in_specs=[pl.BlockSpec((_BLK, _D), lambda i: (i, 0)),
                  pl.BlockSpec((_R, _D, _D), lambda i: (0, 0, 0)),
                  pl.BlockSpec((_D, _D), lambda i: (0, 0))],
        out_specs=[pl.BlockSpec((_R, _BLK, _D), lambda i: (0, i, 0)),
                   pl.BlockSpec((_BLK, _D), lambda i: (i, 0))],
        out_shape=[jax.ShapeDtypeStruct((_R, _N, _D), jnp.float32),
                   jax.ShapeDtypeStruct((_N, _D), jnp.float32)],
    )(h, wrel, wself)


def _combine_body(a0_ref, a1_ref, d0_ref, d1_ref, s_ref, wr_ref, ws_ref,
                  t_ref, s2_ref, h_ref):
    deg = d0_ref[...][:, 0:1] + d1_ref[...][:, 0:1]
    agg = jnp.concatenate([a0_ref[...], a1_ref[...]], axis=1)
    h1 = jnp.maximum(agg / jnp.maximum(deg, 1.0) + s_ref[...], 0.0)
    h_ref[...] = h1
    s2_ref[...] = jnp.dot(h1, ws_ref[...], preferred_element_type=jnp.float32)
    h16 = h1.astype(jnp.bfloat16)
    for r in range(_R):
        t_ref[r] = jnp.dot(h16, wr_ref[r], preferred_element_type=jnp.float32)


def _combine(a0, a1, d0, d1, s1, wrel, wself):
    return pl.pallas_call(
        _combine_body,
        grid=(_NBLK,),
        in_specs=[pl.BlockSpec((_BLK, _HD), lambda i: (i, 0)),
                  pl.BlockSpec((_BLK, _HD), lambda i: (i, 0)),
                  pl.BlockSpec((_BLK, 8), lambda i: (i, 0)),
                  pl.BlockSpec((_BLK, 8), lambda i: (i, 0)),
                  pl.BlockSpec((_BLK, _D), lambda i: (i, 0)),
                  pl.BlockSpec((_R, _D, _D), lambda i: (0, 0, 0)),
                  pl.BlockSpec((_D, _D), lambda i: (0, 0))],
        out_specs=[pl.BlockSpec((_R, _BLK, _D), lambda i: (0, i, 0)),
                   pl.BlockSpec((_BLK, _D), lambda i: (i, 0)),
                   pl.BlockSpec((_BLK, _D), lambda i: (i, 0))],
        out_shape=[jax.ShapeDtypeStruct((_R, _N, _D), jnp.float32),
                   jax.ShapeDtypeStruct((_N, _D), jnp.float32),
                   jax.ShapeDtypeStruct((_N, _D), jnp.float32)],
    )(a0, a1, d0, d1, s1, wrel, wself)


def _readout1_body(h1_ref, gid_ref, hid_ref, tid_ref,
                   g_out, hd_out, tl_out, cnt_out, G, HD, TL, CNT):
    i = pl.program_id(0)

    @pl.when(i == 0)
    def _():
        G[...] = jnp.zeros_like(G)
        HD[...] = jnp.zeros_like(HD)
        TL[...] = jnp.zeros_like(TL)
        CNT[...] = jnp.zeros_like(CNT)

    h1 = h1_ref[...]
    gb = gid_ref[0, 0, :]
    og = (gb[None, :] == lax.broadcasted_iota(jnp.int32, (_B, _BLK), 0)
          ).astype(jnp.float32)
    G[...] += jnp.dot(og, h1, preferred_element_type=jnp.float32)
    CNT[...] = CNT[...] + jnp.sum(og, axis=1, keepdims=True)

    rowid = lax.broadcasted_iota(jnp.int32, (_B, _BLK), 1) + i * _BLK
    oh = (hid_ref[0, :][:, None] == rowid).astype(jnp.float32)
    ot = (tid_ref[0, :][:, None] == rowid).astype(jnp.float32)
    HD[...] += jnp.dot(oh, h1, preferred_element_type=jnp.float32)
    TL[...] += jnp.dot(ot, h1, preferred_element_type=jnp.float32)

    @pl.when(i == _NBLK - 1)
    def _():
        g_out[...] = G[...]
        hd_out[...] = HD[...]
        tl_out[...] = TL[...]
        cnt_out[...] = CNT[...]


def _readout1(h1, gid3, hid2, tid2):
    def cst(*dims):
        return pl.BlockSpec(dims, lambda i: tuple(0 for _ in dims))
    acc = jax.ShapeDtypeStruct((_B, _D), jnp.float32)
    return pl.pallas_call(
        _readout1_body,
        grid=(_NBLK,),
        in_specs=[pl.BlockSpec((_BLK, _D), lambda i: (i, 0)),
                  pl.BlockSpec((1, 1, _BLK), lambda i: (i, 0, 0)),
                  cst(1, _B), cst(1, _B)],
        out_specs=[pl.BlockSpec((_B, _D), lambda i: (0, 0))] * 4,
        out_shape=[acc, acc, acc, acc],
        scratch_shapes=[pltpu.VMEM((_B, _D), jnp.float32)] * 4,
    )(h1, gid3, hid2, tid2)


def _readout2_body(a0_ref, a1_ref, d0_ref, d1_ref, s2_ref, gid_ref,
                   hid_ref, tid_ref, g1_ref, hd1_ref, tl1_ref, cnt_ref,
                   rel_ref, spe_ref, rt_ref,
                   wg1_ref, wg2_ref, wh1_ref, wh2_ref, wt1_ref, wt2_ref,
                   wr_ref, ws_ref, bfc_ref, wo_ref, bo_ref,
                   out_ref, G, HD, TL):
    i = pl.program_id(0)

    @pl.when(i == 0)
    def _():
        G[...] = jnp.zeros_like(G)
        HD[...] = jnp.zeros_like(HD)
        TL[...] = jnp.zeros_like(TL)

    deg = d0_ref[...][:, 0:1] + d1_ref[...][:, 0:1]
    agg = jnp.concatenate([a0_ref[...], a1_ref[...]], axis=1)
    h2 = jnp.maximum(agg / jnp.maximum(deg, 1.0) + s2_ref[...], 0.0)

    gb = gid_ref[0, 0, :]
    og = (gb[None, :] == lax.broadcasted_iota(jnp.int32, (_B, _BLK), 0)
          ).astype(jnp.float32)
    G[...] += jnp.dot(og, h2, preferred_element_type=jnp.float32)

    rowid = lax.broadcasted_iota(jnp.int32, (_B, _BLK), 1) + i * _BLK
    oh = (hid_ref[0, :][:, None] == rowid).astype(jnp.float32)
    ot = (tid_ref[0, :][:, None] == rowid).astype(jnp.float32)
    HD[...] += jnp.dot(oh, h2, preferred_element_type=jnp.float32)
    TL[...] += jnp.dot(ot, h2, preferred_element_type=jnp.float32)

    @pl.when(i == _NBLK - 1)
    def _():
        rcnt = 1.0 / jnp.maximum(cnt_ref[...][:, 0:1], 1.0)
        orl = (rel_ref[0, :][:, None] ==
               lax.broadcasted_iota(jnp.int32, (_B, _R), 1)).astype(jnp.float32)
        relemb = jnp.dot(orl, rt_ref[...], preferred_element_type=jnp.float32)
        hid = (jnp.dot(g1_ref[...] * rcnt, wg1_ref[...],
                       preferred_element_type=jnp.float32)
               + jnp.dot(G[...] * rcnt, wg2_ref[...],
                         preferred_element_type=jnp.float32)
               + jnp.dot(hd1_ref[...], wh1_ref[...],
                         preferred_element_type=jnp.float32)
               + jnp.dot(HD[...], wh2_ref[...],
                         preferred_element_type=jnp.float32)
               + jnp.dot(tl1_ref[...], wt1_ref[...],
                         preferred_element_type=jnp.float32)
               + jnp.dot(TL[...], wt2_ref[...],
                         preferred_element_type=jnp.float32)
               + jnp.dot(relemb, wr_ref[...],
                         preferred_element_type=jnp.float32)
               + jnp.dot(spe_ref[...], ws_ref[...],
                         preferred_element_type=jnp.float32)
               + bfc_ref[...])
        hid = jnp.maximum(hid, 0.0)
        res = jnp.sum(hid * wo_ref[...], axis=1, keepdims=True) + bo_ref[0, 0]
        out_ref[...] = jnp.broadcast_to(res, (_B, _D))


def _readout2(a0, a1, d0, d1, s2, gid3, hid2, tid2, g1, hd1, tl1, cnt,
              rel2, spe, rtab, wg1, wg2, wh1, wh2, wt1, wt2, wr, ws,
              bfc2, wo2, bo2):
    def cst(*dims):
        return pl.BlockSpec(dims, lambda i: tuple(0 for _ in dims))
    return pl.pallas_call(
        _readout2_body,
        grid=(_NBLK,),
        in_specs=[pl.BlockSpec((_BLK, _HD), lambda i: (i, 0)),
                  pl.BlockSpec((_BLK, _HD), lambda i: (i, 0)),
                  pl.BlockSpec((_BLK, 8), lambda i: (i, 0)),
                  pl.BlockSpec((_BLK, 8), lambda i: (i, 0)),
                  pl.BlockSpec((_BLK, _D), lambda i: (i, 0)),
                  pl.BlockSpec((1, 1, _BLK), lambda i: (i, 0, 0)),
                  cst(1, _B), cst(1, _B),
                  cst(_B, _D), cst(_B, _D), cst(_B, _D), cst(_B, _D),
                  cst(1, _B),
                  cst(_B, 16), cst(_R, _R),
                  cst(_D, _H), cst(_D, _H), cst(_D, _H), cst(_D, _H),
                  cst(_D, _H), cst(_D, _H),
                  cst(_R, _H), cst(16, _H), cst(1, _H), cst(1, _H),
                  cst(1, 1)],
        out_specs=pl.BlockSpec((_B, _D), lambda i: (0, 0)),
        out_shape=jax.ShapeDtypeStruct((_B, _D), jnp.float32),
        scratch_shapes=[pltpu.VMEM((_B, _D), jnp.float32)] * 3,
    )(a0, a1, d0, d1, s2, gid3, hid2, tid2, g1, hd1, tl1, cnt,
      rel2, spe, rtab, wg1, wg2, wh1, wh2, wt1, wt2, wr, ws,
      bfc2, wo2, bo2)


def kernel(x, edge_index, edge_type, graph_ids, head_ids, tail_ids,
           rel_labels, spe, W_rel1, W_self1, W_rel2, W_self2, rel_table,
           W_fc, b_fc, W_out, b_out):
    src3 = edge_index[0].astype(jnp.int32).reshape(_NS, _EPT)
    dst3 = edge_index[1].astype(jnp.int32).reshape(_NS, _EPT)
    ety3 = edge_type.astype(jnp.int32).reshape(_NS, _EPT)

    T1, S1 = _transform(x, W_rel1.astype(jnp.bfloat16), W_self1)
    a0, a1, d0, d1 = _sc_agg_deg(T1.reshape(_N * _R * 2, _HD),
                                 src3, ety3, dst3,
                                 jnp.ones((_CHUNK, 8), jnp.float32),
                                 jnp.zeros((128, 8), jnp.float32))
    T2, S2, H1 = _combine(a0, a1, d0, d1, S1,
                          W_rel2.astype(jnp.bfloat16), W_self2)
    b0, b1 = _sc_agg(T2.reshape(_N * _R * 2, _HD), src3, ety3, dst3)

    gid3 = graph_ids.astype(jnp.int32).reshape(_NBLK, 1, _BLK)
    hid2 = head_ids.astype(jnp.int32).reshape(1, _B)
    tid2 = tail_ids.astype(jnp.int32).reshape(1, _B)
    G1, HD1, TL1, CNT = _readout1(H1, gid3, hid2, tid2)

    OUT = _readout2(b0, b1, d0, d1, S2, gid3, hid2, tid2, G1, HD1, TL1, CNT,
                    rel_labels.astype(jnp.int32).reshape(1, _B),
                    spe, rel_table,
                    W_fc[0:_D], W_fc[_D:2 * _D], W_fc[2 * _D:3 * _D],
                    W_fc[3 * _D:4 * _D], W_fc[4 * _D:5 * _D],
                    W_fc[5 * _D:6 * _D],
                    W_fc[6 * _D:6 * _D + _R], W_fc[6 * _D + _R:],
                    b_fc.reshape(1, _H), W_out.reshape(1, _H),
                    b_out.reshape(1, 1))
    return OUT[:, :1]
